# Initial kernel scaffold; baseline (speedup 1.0000x reference)
import functools

import jax
import jax.numpy as jnp
from jax import lax
from jax.experimental import pallas as pl
from jax.experimental.pallas import tpu as pltpu
from jax.experimental.pallas import tpu_sc as plsc

N = 10000
E = 320000
DN = 128
DE = 4
LAT = 6
H = 16
NCONV = 8

NC = 2
NS = 16
NW = NC * NS
NPAD = N + 16
SUP = 1024
CHB = SUP // 128
EPW = 10240
E_PAD = EPW * NW
NSUP = EPW // SUP
RPT = NPAD // NS

F32 = jnp.float32


def _mesh():
    return plsc.VectorSubcoreMesh(
        core_axis_name="c", subcore_axis_name="s", num_cores=NC, num_subcores=NS
    )


def _edge_body_common(a_hbm, b_hbm, ew_hbm, src_hbm, dst_hbm, accp_hbm,
                      cntp_hbm, sidx, didx, abuf, bbuf, ewbuf, zbuf, obuf,
                      acc_sh, cnt_sh, sem_a, sem_b, sem_m, with_b, with_cnt):
    c = lax.axis_index("c")
    s = lax.axis_index("s")
    wid = c * NS + s

    zero16 = jnp.zeros((16,), F32)

    def zrow(r, carry):
        zbuf[r, :] = zero16
        return carry

    lax.fori_loop(0, RPT, zrow, 0)
    pltpu.sync_copy(zbuf, acc_sh.at[pl.ds(s * RPT, RPT)])
    if with_cnt:
        lane = lax.iota(jnp.int32, 16)
        onerow = jnp.where(lane == 0, 1.0, 0.0).astype(F32)

        def orow(r, carry):
            obuf[r, :] = onerow
            return carry

        lax.fori_loop(0, 128, orow, 0)
        pltpu.sync_copy(zbuf, cnt_sh.at[pl.ds(s * RPT, RPT)])
    plsc.subcore_barrier()

    base0 = wid * EPW

    def sup_body(g, carry):
        base = base0 + g * SUP
        pltpu.sync_copy(src_hbm.at[pl.ds(base, SUP)], sidx)
        for j in range(CHB):
            pltpu.sync_copy(dst_hbm.at[pl.ds(base + j * 128, 128)], didx.at[j])
        dm = pltpu.async_copy(ew_hbm.at[pl.ds(base, SUP)], ewbuf, sem_m)
        da = [
            pltpu.async_copy(
                a_hbm.at[sidx.at[pl.ds(j * 128, 128)]],
                abuf.at[pl.ds(j * 128, 128)], sem_a)
            for j in range(CHB)
        ]
        db = []
        if with_b:
            db = [
                pltpu.async_copy(
                    b_hbm.at[didx.at[j]],
                    bbuf.at[pl.ds(j * 128, 128)], sem_b)
                for j in range(CHB)
            ]
        dm.wait()
        for d in da:
            d.wait()
        for d in db:
            d.wait()

        if with_b:
            def comp(e, carry2):
                abuf[e] = jnp.maximum(abuf[e] + bbuf[e] + ewbuf[e], 0.0)
                return carry2
        else:
            def comp(e, carry2):
                abuf[e] = jnp.maximum(abuf[e] + ewbuf[e], 0.0)
                return carry2

        lax.fori_loop(0, SUP, comp, 0)

        for j in range(CHB):
            pltpu.sync_copy(abuf.at[pl.ds(j * 128, 128)],
                            acc_sh.at[didx.at[j]], add=True)
        if with_cnt:
            for j in range(CHB):
                pltpu.sync_copy(obuf, cnt_sh.at[didx.at[j]], add=True)
        return carry

    lax.fori_loop(0, NSUP, sup_body, 0)
    plsc.subcore_barrier()
    pltpu.sync_copy(acc_sh.at[pl.ds(s * RPT, RPT)],
                    accp_hbm.at[c, pl.ds(s * RPT, RPT)])
    if with_cnt:
        pltpu.sync_copy(cnt_sh.at[pl.ds(s * RPT, RPT)],
                        cntp_hbm.at[c, pl.ds(s * RPT, RPT)])


def _make_edge0():
    out_type = (
        jax.ShapeDtypeStruct((NC, NPAD, 16), F32),
        jax.ShapeDtypeStruct((NC, NPAD, 16), F32),
    )
    scratch = [
        pltpu.VMEM((SUP,), jnp.int32),
        pltpu.VMEM((CHB, 128), jnp.int32),
        pltpu.VMEM((SUP, 16), F32),
        pltpu.VMEM((SUP, 16), F32),
        pltpu.VMEM((RPT, 16), F32),
        pltpu.VMEM((128, 16), F32),
        pltpu.VMEM_SHARED((NPAD, 16), F32),
        pltpu.VMEM_SHARED((NPAD, 16), F32),
        pltpu.SemaphoreType.DMA,
        pltpu.SemaphoreType.DMA,
    ]

    @functools.partial(pl.kernel, out_type=out_type, mesh=_mesh(),
                       scratch_types=scratch)
    def k(a_hbm, ew_hbm, src_hbm, dst_hbm, accp_hbm, cntp_hbm,
          sidx, didx, abuf, ewbuf, zbuf, obuf, acc_sh, cnt_sh, sem_a, sem_m):
        _edge_body_common(a_hbm, None, ew_hbm, src_hbm, dst_hbm, accp_hbm,
                          cntp_hbm, sidx, didx, abuf, None, ewbuf, zbuf, obuf,
                          acc_sh, cnt_sh, sem_a, None, sem_m,
                          with_b=False, with_cnt=True)

    return k


def _make_edge():
    out_type = jax.ShapeDtypeStruct((NC, NPAD, 16), F32)
    scratch = [
        pltpu.VMEM((SUP,), jnp.int32),
        pltpu.VMEM((CHB, 128), jnp.int32),
        pltpu.VMEM((SUP, 16), F32),
        pltpu.VMEM((SUP, 16), F32),
        pltpu.VMEM((SUP, 16), F32),
        pltpu.VMEM((RPT, 16), F32),
        pltpu.VMEM_SHARED((NPAD, 16), F32),
        pltpu.SemaphoreType.DMA,
        pltpu.SemaphoreType.DMA,
        pltpu.SemaphoreType.DMA,
    ]

    @functools.partial(pl.kernel, out_type=out_type, mesh=_mesh(),
                       scratch_types=scratch)
    def k(a_hbm, b_hbm, ew_hbm, src_hbm, dst_hbm, accp_hbm,
          sidx, didx, abuf, bbuf, ewbuf, zbuf, acc_sh, sem_a, sem_b, sem_m):
        _edge_body_common(a_hbm, b_hbm, ew_hbm, src_hbm, dst_hbm, accp_hbm,
                          None, sidx, didx, abuf, bbuf, ewbuf, zbuf, None,
                          acc_sh, None, sem_a, sem_b, sem_m,
                          with_b=True, with_cnt=False)

    return k


def _make_balance():
    out_type = (
        jax.ShapeDtypeStruct((E_PAD,), F32),
        jax.ShapeDtypeStruct((NW, NPAD), F32),
    )
    scratch = [
        pltpu.VMEM((NPAD,), F32),
        pltpu.VMEM((NPAD,), F32),
        pltpu.VMEM((SUP,), jnp.int32),
        pltpu.VMEM((SUP,), jnp.int32),
        pltpu.VMEM((SUP,), F32),
        pltpu.VMEM((SUP,), F32),
    ]

    @functools.partial(pl.kernel, out_type=out_type, mesh=_mesh(),
                       scratch_types=scratch)
    def k(p_hbm, src_hbm, dst_hbm, ea_hbm, flow_hbm, netp_hbm,
          ptab, netacc, sbuf, dbuf, eabuf, fbuf):
        c = lax.axis_index("c")
        s = lax.axis_index("s")
        wid = c * NS + s
        pltpu.sync_copy(p_hbm, ptab)
        zero16 = jnp.zeros((16,), F32)

        def zr(r, carry):
            netacc[pl.ds(r * 16, 16)] = zero16
            return carry

        lax.fori_loop(0, NPAD // 16, zr, 0)

        base0 = wid * EPW

        def sup_body(g, carry):
            base = base0 + g * SUP
            pltpu.sync_copy(src_hbm.at[pl.ds(base, SUP)], sbuf)
            pltpu.sync_copy(dst_hbm.at[pl.ds(base, SUP)], dbuf)
            pltpu.sync_copy(ea_hbm.at[pl.ds(base, SUP)], eabuf)

            def v16(kk, carry2):
                sv = sbuf[pl.ds(kk * 16, 16)]
                dv = dbuf[pl.ds(kk * 16, 16)]
                ps = plsc.load_gather(ptab, [sv])
                pd = plsc.load_gather(ptab, [dv])
                fl = (ps - pd) * eabuf[pl.ds(kk * 16, 16)]
                fbuf[pl.ds(kk * 16, 16)] = fl
                return carry2

            lax.fori_loop(0, SUP // 16, v16, 0)
            pltpu.sync_copy(fbuf, flow_hbm.at[pl.ds(base, SUP)])

            def sc(e, carry2):
                si = sbuf[e]
                di = dbuf[e]
                f = fbuf[e]
                netacc[si] = netacc[si] - f
                netacc[di] = netacc[di] + f
                return carry2

            lax.fori_loop(0, SUP, sc, 0)
            return carry

        lax.fori_loop(0, NSUP, sup_body, 0)
        pltpu.sync_copy(netacc, netp_hbm.at[wid])

    return k


_NBLK = 2504
_NGRID = NPAD // _NBLK
_EBLK = 4096
_EGRID = E_PAD // _EBLK


def _prep_kernel(x_ref, wx_ref, b1_ref, xw_ref, a0_ref):
    xw = jnp.dot(x_ref[...], wx_ref[...], preferred_element_type=F32)
    xw_ref[...] = xw
    a0_ref[...] = xw[:, 0:16] + b1_ref[0:1, :]


def _prep_call(x_pad, wx, b1_0):
    return pl.pallas_call(
        _prep_kernel,
        grid=(_NGRID,),
        in_specs=[
            pl.BlockSpec((_NBLK, DN), lambda i: (i, 0)),
            pl.BlockSpec((DN, DN), lambda i: (0, 0)),
            pl.BlockSpec((8, 16), lambda i: (0, 0)),
        ],
        out_specs=[
            pl.BlockSpec((_NBLK, DN), lambda i: (i, 0)),
            pl.BlockSpec((_NBLK, 16), lambda i: (i, 0)),
        ],
        out_shape=[
            jax.ShapeDtypeStruct((NPAD, DN), F32),
            jax.ShapeDtypeStruct((NPAD, 16), F32),
        ],
    )(x_pad, wx, b1_0)


def _ew_kernel(ea_ref, we_ref, *out_refs):
    r = jnp.dot(ea_ref[...], we_ref[0:DE, :], preferred_element_type=F32)
    for i, o in enumerate(out_refs):
        o[...] = r[:, 16 * i:16 * (i + 1)]


def _ew_call(ea_pad, we):
    return pl.pallas_call(
        _ew_kernel,
        grid=(_EGRID,),
        in_specs=[
            pl.BlockSpec((_EBLK, DE), lambda i: (i, 0)),
            pl.BlockSpec((8, DN), lambda i: (0, 0)),
        ],
        out_specs=[pl.BlockSpec((_EBLK, 16), lambda i: (i, 0))
                   for _ in range(NCONV)],
        out_shape=[jax.ShapeDtypeStruct((E_PAD, 16), F32)
                   for _ in range(NCONV)],
    )(ea_pad, we)


def _node_common(accp, w2, b2):
    s = accp[0] + accp[1]
    return jnp.dot(s, w2[...], preferred_element_type=F32), b2[0:1, 0:LAT]


def _node0_kernel(accp_ref, cntp_ref, xw_ref, w2_ref, b2_ref, w1s_ref,
                  w1d_ref, b1_ref, a_ref, b_ref, cnt0_ref, cntc_ref):
    sw2, b2 = _node_common(accp_ref[...], w2_ref, b2_ref)
    cnt0 = (cntp_ref[0, :, 0] + cntp_ref[1, :, 0]).reshape(-1, 1)
    cntc = jnp.maximum(cnt0, 1.0)
    agg = (sw2 + cnt0 * b2) / cntc
    xx = jnp.maximum(agg, 0.0)
    a_ref[...] = (jnp.dot(xx, w1s_ref[0:LAT, :], preferred_element_type=F32)
                  + xw_ref[:, 16:32] + b1_ref[0:1, :])
    b_ref[...] = jnp.dot(xx, w1d_ref[0:LAT, :], preferred_element_type=F32)
    cnt0_ref[...] = cnt0
    cntc_ref[...] = cntc


def _node_mid_kernel(i, accp_ref, cnt0_ref, cntc_ref, xw_ref, w2_ref, b2_ref,
                     w1s_ref, w1d_ref, b1_ref, a_ref, b_ref):
    sw2, b2 = _node_common(accp_ref[...], w2_ref, b2_ref)
    agg = (sw2 + cnt0_ref[...] * b2) / cntc_ref[...]
    xx = jnp.maximum(agg, 0.0)
    a_ref[...] = (jnp.dot(xx, w1s_ref[0:LAT, :], preferred_element_type=F32)
                  + xw_ref[:, 16 * (i + 1):16 * (i + 2)] + b1_ref[0:1, :])
    b_ref[...] = jnp.dot(xx, w1d_ref[0:LAT, :], preferred_element_type=F32)


def _node_last_kernel(accp_ref, cnt0_ref, cntc_ref, w2_ref, b2_ref, wf_ref,
                      bf_ref, xlast_ref, p_ref, pm_ref):
    sw2, b2 = _node_common(accp_ref[...], w2_ref, b2_ref)
    agg = (sw2 + cnt0_ref[...] * b2) / cntc_ref[...]
    xx = jnp.maximum(agg, 0.0)
    p = jnp.maximum(
        jnp.dot(xx, wf_ref[0:LAT, :], preferred_element_type=F32)
        + bf_ref[0:1, :], 0.0)
    p_ref[...] = p
    xl = xlast_ref[...]
    pm_ref[...] = jnp.where(xl != 0.0, xl, p)


def _node0_call(accp, cntp, xw, w2, b2, w1s, w1d, b1):
    full = lambda shape: pl.BlockSpec(shape, lambda i: tuple(0 for _ in shape))
    return pl.pallas_call(
        _node0_kernel,
        grid=(_NGRID,),
        in_specs=[
            pl.BlockSpec((NC, _NBLK, 16), lambda i: (0, i, 0)),
            pl.BlockSpec((NC, _NBLK, 16), lambda i: (0, i, 0)),
            pl.BlockSpec((_NBLK, DN), lambda i: (i, 0)),
            full((16, LAT)), full((8, 8)), full((8, 16)), full((8, 16)),
            full((8, 16)),
        ],
        out_specs=[
            pl.BlockSpec((_NBLK, 16), lambda i: (i, 0)),
            pl.BlockSpec((_NBLK, 16), lambda i: (i, 0)),
            pl.BlockSpec((_NBLK, 1), lambda i: (i, 0)),
            pl.BlockSpec((_NBLK, 1), lambda i: (i, 0)),
        ],
        out_shape=[
            jax.ShapeDtypeStruct((NPAD, 16), F32),
            jax.ShapeDtypeStruct((NPAD, 16), F32),
            jax.ShapeDtypeStruct((NPAD, 1), F32),
            jax.ShapeDtypeStruct((NPAD, 1), F32),
        ],
    )(accp, cntp, xw, w2, b2, w1s, w1d, b1)


def _node_mid_call(i, accp, cnt0, cntc, xw, w2, b2, w1s, w1d, b1):
    full = lambda shape: pl.BlockSpec(shape, lambda i_: tuple(0 for _ in shape))
    return pl.pallas_call(
        functools.partial(_node_mid_kernel, i),
        grid=(_NGRID,),
        in_specs=[
            pl.BlockSpec((NC, _NBLK, 16), lambda i_: (0, i_, 0)),
            pl.BlockSpec((_NBLK, 1), lambda i_: (i_, 0)),
            pl.BlockSpec((_NBLK, 1), lambda i_: (i_, 0)),
            pl.BlockSpec((_NBLK, DN), lambda i_: (i_, 0)),
            full((16, LAT)), full((8, 8)), full((8, 16)), full((8, 16)),
            full((8, 16)),
        ],
        out_specs=[
            pl.BlockSpec((_NBLK, 16), lambda i_: (i_, 0)),
            pl.BlockSpec((_NBLK, 16), lambda i_: (i_, 0)),
        ],
        out_shape=[
            jax.ShapeDtypeStruct((NPAD, 16), F32),
            jax.ShapeDtypeStruct((NPAD, 16), F32),
        ],
    )(accp, cnt0, cntc, xw, w2, b2, w1s, w1d, b1)


def _node_last_call(accp, cnt0, cntc, w2, b2, wf, bf, xlast):
    full = lambda shape: pl.BlockSpec(shape, lambda i: tuple(0 for _ in shape))
    return pl.pallas_call(
        _node_last_kernel,
        grid=(_NGRID,),
        in_specs=[
            pl.BlockSpec((NC, _NBLK, 16), lambda i: (0, i, 0)),
            pl.BlockSpec((_NBLK, 1), lambda i: (i, 0)),
            pl.BlockSpec((_NBLK, 1), lambda i: (i, 0)),
            full((16, LAT)), full((8, 8)), full((8, 1)), full((8, 1)),
            pl.BlockSpec((_NBLK, 1), lambda i: (i, 0)),
        ],
        out_specs=[
            pl.BlockSpec((_NBLK, 1), lambda i: (i, 0)),
            pl.BlockSpec((_NBLK, 1), lambda i: (i, 0)),
        ],
        out_shape=[
            jax.ShapeDtypeStruct((NPAD, 1), F32),
            jax.ShapeDtypeStruct((NPAD, 1), F32),
        ],
    )(accp, cnt0, cntc, w2, b2, wf, bf, xlast)


def _imbal_kernel(netp_ref, p_ref, out_ref):
    net = p_ref[0:1, :] + jnp.sum(netp_ref[...], axis=0, keepdims=True)
    out_ref[0, 0] = jnp.sum(jnp.abs(net))


def _imbal_call(netp, p_row):
    return pl.pallas_call(
        _imbal_kernel,
        grid=(1,),
        in_specs=[
            pl.BlockSpec((NW, NPAD), lambda i: (0, 0)),
            pl.BlockSpec((8, NPAD), lambda i: (0, 0)),
        ],
        out_specs=pl.BlockSpec((1, 1), lambda i: (0, 0)),
        out_shape=jax.ShapeDtypeStruct((1, 1), F32),
    )(netp, p_row)


def kernel(x, edge_index, edge_attr, W1, b1, W2, b2, Wf, bf):
    src = edge_index[0]
    dst = edge_index[1]
    npad_e = E_PAD - E
    src_p = jnp.concatenate([src, jnp.zeros((npad_e,), jnp.int32)])
    dst_p = jnp.concatenate([dst, jnp.full((npad_e,), N, jnp.int32)])
    ea_p = jnp.concatenate([edge_attr, jnp.zeros((npad_e, DE), F32)], axis=0)
    x_pad = jnp.concatenate([x, jnp.zeros((16, DN), F32)], axis=0)
    xlast_pad = x_pad[:, DN - 1:DN]

    w1s = W1[:, 0:LAT, :]
    w1d = W1[:, LAT:2 * LAT, :]
    w1x = W1[:, 2 * LAT:2 * LAT + DN, :]
    w1e = W1[:, 2 * LAT + DN:, :]
    wx_all = jnp.transpose(w1x, (1, 0, 2)).reshape(DN, NCONV * 16)
    we_all = jnp.concatenate([
        jnp.transpose(w1e, (1, 0, 2)).reshape(DE, NCONV * 16),
        jnp.zeros((8 - DE, NCONV * 16), F32)], axis=0)
    w1s_p = jnp.concatenate([w1s, jnp.zeros((NCONV, 2, 16), F32)], axis=1)
    w1d_p = jnp.concatenate([w1d, jnp.zeros((NCONV, 2, 16), F32)], axis=1)
    b1_bc = jnp.broadcast_to(b1[:, None, :], (NCONV, 8, 16))
    b2_bc = jnp.broadcast_to(
        jnp.pad(b2, ((0, 0), (0, 2)))[:, None, :], (NCONV, 8, 8))
    wf_p = jnp.concatenate([Wf, jnp.zeros((2, 1), F32)], axis=0)
    bf_bc = jnp.broadcast_to(bf.reshape(1, 1), (8, 1))

    xw, a0 = _prep_call(x_pad, wx_all, b1_bc[0])
    ews = _ew_call(ea_p, we_all)

    edge0 = _make_edge0()
    edge = _make_edge()

    accp, cntp = edge0(a0, ews[0], src_p, dst_p)
    a_t, b_t, cnt0, cntc = _node0_call(
        accp, cntp, xw, W2[0], b2_bc[0], w1s_p[1], w1d_p[1], b1_bc[1])

    for i in range(1, NCONV - 1):
        accp = edge(a_t, b_t, ews[i], src_p, dst_p)
        a_t, b_t = _node_mid_call(
            i, accp, cnt0, cntc, xw, W2[i], b2_bc[i],
            w1s_p[i + 1], w1d_p[i + 1], b1_bc[i + 1])

    accp = edge(a_t, b_t, ews[NCONV - 1], src_p, dst_p)
    p_full, pm_full = _node_last_call(
        accp, cnt0, cntc, W2[NCONV - 1], b2_bc[NCONV - 1], wf_p, bf_bc,
        xlast_pad)

    p_tab = jnp.concatenate([pm_full[:N, 0], jnp.zeros((16,), F32)])
    ea0_p = ea_p[:, 0]

    balance = _make_balance()
    flow_pad, netp = balance(p_tab, src_p, dst_p, ea0_p)

    p_row = jnp.broadcast_to(p_tab[None, :], (8, NPAD))
    imb = _imbal_call(netp, p_row)

    P = p_full[:N]
    flow = flow_pad[:E].reshape(E, 1)
    return (P, flow, imb.reshape(1))


# speedup vs baseline: 8.3109x; 8.3109x over previous
import functools

import jax
import jax.numpy as jnp
from jax import lax
from jax.experimental import pallas as pl
from jax.experimental.pallas import tpu as pltpu
from jax.experimental.pallas import tpu_sc as plsc

N = 10000
E = 320000
DN = 128
DE = 4
LAT = 6
H = 16
NCONV = 8

NC = 2
NS = 16
NW = NC * NS
NPAD = N + 112
SUP = 1024
CHB = SUP // 128
EPW = 10240
E_PAD = EPW * NW
NSUP = EPW // SUP
RPT = NPAD // NS

F32 = jnp.float32


def _mesh():
    return plsc.VectorSubcoreMesh(
        core_axis_name="c", subcore_axis_name="s", num_cores=NC, num_subcores=NS
    )


def _edge_body_common(a_hbm, b_hbm, ew_hbm, src_hbm, dst_hbm, accp_hbm,
                      cntp_hbm, sidx, didx, abuf, bbuf, ewbuf, zbuf, obuf,
                      acc_sh, cnt_sh, sem_a, sem_b, sem_m, with_b, with_cnt):
    c = lax.axis_index("c")
    s = lax.axis_index("s")
    wid = c * NS + s

    zero16 = jnp.zeros((16,), F32)

    def zrow(r, carry):
        zbuf[r, :] = zero16
        return carry

    lax.fori_loop(0, RPT, zrow, 0)
    pltpu.sync_copy(zbuf, acc_sh.at[pl.ds(s * RPT, RPT)])
    if with_cnt:
        lane = lax.iota(jnp.int32, 16)
        onerow = jnp.where(lane == 0, 1.0, 0.0).astype(F32)

        def orow(r, carry):
            obuf[r, :] = onerow
            return carry

        lax.fori_loop(0, 128, orow, 0)
        pltpu.sync_copy(zbuf, cnt_sh.at[pl.ds(s * RPT, RPT)])
    plsc.subcore_barrier()

    base0 = wid * EPW

    def sup_body(g, carry):
        base = base0 + g * SUP
        pltpu.sync_copy(src_hbm.at[pl.ds(base, SUP)], sidx)
        for j in range(CHB):
            pltpu.sync_copy(dst_hbm.at[pl.ds(base + j * 128, 128)], didx.at[j])
        dm = pltpu.async_copy(ew_hbm.at[pl.ds(base, SUP)], ewbuf, sem_m)
        da = [
            pltpu.async_copy(
                a_hbm.at[sidx.at[pl.ds(j * 128, 128)]],
                abuf.at[pl.ds(j * 128, 128)], sem_a)
            for j in range(CHB)
        ]
        db = []
        if with_b:
            db = [
                pltpu.async_copy(
                    b_hbm.at[didx.at[j]],
                    bbuf.at[pl.ds(j * 128, 128)], sem_b)
                for j in range(CHB)
            ]
        dm.wait()
        for d in da:
            d.wait()
        for d in db:
            d.wait()

        if with_b:
            def comp(e, carry2):
                abuf[e] = jnp.maximum(abuf[e] + bbuf[e] + ewbuf[e], 0.0)
                return carry2
        else:
            def comp(e, carry2):
                abuf[e] = jnp.maximum(abuf[e] + ewbuf[e], 0.0)
                return carry2

        lax.fori_loop(0, SUP, comp, 0)

        for j in range(CHB):
            pltpu.sync_copy(abuf.at[pl.ds(j * 128, 128)],
                            acc_sh.at[didx.at[j]], add=True)
        if with_cnt:
            for j in range(CHB):
                pltpu.sync_copy(obuf, cnt_sh.at[didx.at[j]], add=True)
        return carry

    lax.fori_loop(0, NSUP, sup_body, 0)
    plsc.subcore_barrier()
    pltpu.sync_copy(acc_sh.at[pl.ds(s * RPT, RPT)],
                    accp_hbm.at[c, pl.ds(s * RPT, RPT)])
    if with_cnt:
        pltpu.sync_copy(cnt_sh.at[pl.ds(s * RPT, RPT)],
                        cntp_hbm.at[c, pl.ds(s * RPT, RPT)])


def _make_edge0():
    out_type = (
        jax.ShapeDtypeStruct((NC, NPAD, 16), F32),
        jax.ShapeDtypeStruct((NC, NPAD, 16), F32),
    )
    scratch = [
        pltpu.VMEM((SUP,), jnp.int32),
        pltpu.VMEM((CHB, 128), jnp.int32),
        pltpu.VMEM((SUP, 16), F32),
        pltpu.VMEM((SUP, 16), F32),
        pltpu.VMEM((RPT, 16), F32),
        pltpu.VMEM((128, 16), F32),
        pltpu.VMEM_SHARED((NPAD, 16), F32),
        pltpu.VMEM_SHARED((NPAD, 16), F32),
        pltpu.SemaphoreType.DMA,
        pltpu.SemaphoreType.DMA,
    ]

    @functools.partial(pl.kernel, out_type=out_type, mesh=_mesh(),
                       scratch_types=scratch,
                       compiler_params=pltpu.CompilerParams(
                           use_tc_tiling_on_sc=False, needs_layout_passes=False))
    def k(a_hbm, ew_hbm, src_hbm, dst_hbm, accp_hbm, cntp_hbm,
          sidx, didx, abuf, ewbuf, zbuf, obuf, acc_sh, cnt_sh, sem_a, sem_m):
        _edge_body_common(a_hbm, None, ew_hbm, src_hbm, dst_hbm, accp_hbm,
                          cntp_hbm, sidx, didx, abuf, None, ewbuf, zbuf, obuf,
                          acc_sh, cnt_sh, sem_a, None, sem_m,
                          with_b=False, with_cnt=True)

    return k


def _make_edge():
    out_type = jax.ShapeDtypeStruct((NC, NPAD, 16), F32)
    scratch = [
        pltpu.VMEM((SUP,), jnp.int32),
        pltpu.VMEM((CHB, 128), jnp.int32),
        pltpu.VMEM((SUP, 16), F32),
        pltpu.VMEM((SUP, 16), F32),
        pltpu.VMEM((SUP, 16), F32),
        pltpu.VMEM((RPT, 16), F32),
        pltpu.VMEM_SHARED((NPAD, 16), F32),
        pltpu.SemaphoreType.DMA,
        pltpu.SemaphoreType.DMA,
        pltpu.SemaphoreType.DMA,
    ]

    @functools.partial(pl.kernel, out_type=out_type, mesh=_mesh(),
                       scratch_types=scratch,
                       compiler_params=pltpu.CompilerParams(
                           use_tc_tiling_on_sc=False, needs_layout_passes=False))
    def k(a_hbm, b_hbm, ew_hbm, src_hbm, dst_hbm, accp_hbm,
          sidx, didx, abuf, bbuf, ewbuf, zbuf, acc_sh, sem_a, sem_b, sem_m):
        _edge_body_common(a_hbm, b_hbm, ew_hbm, src_hbm, dst_hbm, accp_hbm,
                          None, sidx, didx, abuf, bbuf, ewbuf, zbuf, None,
                          acc_sh, None, sem_a, sem_b, sem_m,
                          with_b=True, with_cnt=False)

    return k


def _make_balance():
    out_type = (
        jax.ShapeDtypeStruct((E_PAD,), F32),
        jax.ShapeDtypeStruct((NW, NPAD), F32),
    )
    scratch = [
        pltpu.VMEM((NPAD,), F32),
        pltpu.VMEM((NPAD,), F32),
        pltpu.VMEM((SUP,), jnp.int32),
        pltpu.VMEM((SUP,), jnp.int32),
        pltpu.VMEM((SUP,), F32),
        pltpu.VMEM((SUP,), F32),
    ]

    @functools.partial(pl.kernel, out_type=out_type, mesh=_mesh(),
                       scratch_types=scratch,
                       compiler_params=pltpu.CompilerParams(
                           use_tc_tiling_on_sc=False, needs_layout_passes=False))
    def k(p_hbm, src_hbm, dst_hbm, ea_hbm, flow_hbm, netp_hbm,
          ptab, netacc, sbuf, dbuf, eabuf, fbuf):
        c = lax.axis_index("c")
        s = lax.axis_index("s")
        wid = c * NS + s
        pltpu.sync_copy(p_hbm, ptab)
        zero16 = jnp.zeros((16,), F32)

        def zr(r, carry):
            netacc[pl.ds(r * 16, 16)] = zero16
            return carry

        lax.fori_loop(0, NPAD // 16, zr, 0)

        base0 = wid * EPW
        lane = lax.iota(jnp.int32, 16)

        def sup_body(g, carry):
            base = base0 + g * SUP
            pltpu.sync_copy(src_hbm.at[pl.ds(base, SUP)], sbuf)
            pltpu.sync_copy(dst_hbm.at[pl.ds(base, SUP)], dbuf)
            pltpu.sync_copy(ea_hbm.at[pl.ds(base, SUP)], eabuf)

            def v16(kk, carry2):
                sv = sbuf[pl.ds(kk * 16, 16)]
                dv = dbuf[pl.ds(kk * 16, 16)]
                ps = plsc.load_gather(ptab, [sv])
                pd = plsc.load_gather(ptab, [dv])
                fl = (ps - pd) * eabuf[pl.ds(kk * 16, 16)]
                fbuf[pl.ds(kk * 16, 16)] = fl
                nfl = -fl
                for j in range(16):
                    m = lane == j
                    plsc.addupdate_scatter(netacc, [sv], nfl, mask=m)
                    plsc.addupdate_scatter(netacc, [dv], fl, mask=m)
                return carry2

            lax.fori_loop(0, SUP // 16, v16, 0)
            pltpu.sync_copy(fbuf, flow_hbm.at[pl.ds(base, SUP)])
            return carry

        lax.fori_loop(0, NSUP, sup_body, 0)
        pltpu.sync_copy(netacc, netp_hbm.at[wid])

    return k


_NBLK = NPAD // 4
_NGRID = NPAD // _NBLK
_EBLK = 4096
_EGRID = E_PAD // _EBLK


def _prep_kernel(x_ref, wx_ref, b1_ref, xw_ref, a0_ref):
    xw = jnp.dot(x_ref[...], wx_ref[...], preferred_element_type=F32)
    xw_ref[...] = xw
    a0_ref[...] = xw[:, 0:16] + b1_ref[0:1, :]


def _prep_call(x_pad, wx, b1_0):
    return pl.pallas_call(
        _prep_kernel,
        grid=(_NGRID,),
        in_specs=[
            pl.BlockSpec((_NBLK, DN), lambda i: (i, 0)),
            pl.BlockSpec((DN, DN), lambda i: (0, 0)),
            pl.BlockSpec((8, 16), lambda i: (0, 0)),
        ],
        out_specs=[
            pl.BlockSpec((_NBLK, DN), lambda i: (i, 0)),
            pl.BlockSpec((_NBLK, 16), lambda i: (i, 0)),
        ],
        out_shape=[
            jax.ShapeDtypeStruct((NPAD, DN), F32),
            jax.ShapeDtypeStruct((NPAD, 16), F32),
        ],
    )(x_pad, wx, b1_0)


def _ew_kernel(ea_ref, we_ref, *out_refs):
    r = jnp.dot(ea_ref[...], we_ref[0:DE, :], preferred_element_type=F32)
    for i, o in enumerate(out_refs):
        o[...] = r[:, 16 * i:16 * (i + 1)]


def _ew_call(ea_pad, we):
    return pl.pallas_call(
        _ew_kernel,
        grid=(_EGRID,),
        in_specs=[
            pl.BlockSpec((_EBLK, DE), lambda i: (i, 0)),
            pl.BlockSpec((8, DN), lambda i: (0, 0)),
        ],
        out_specs=[pl.BlockSpec((_EBLK, 16), lambda i: (i, 0))
                   for _ in range(NCONV)],
        out_shape=[jax.ShapeDtypeStruct((E_PAD, 16), F32)
                   for _ in range(NCONV)],
    )(ea_pad, we)


def _node_common(accp, w2, b2):
    s = accp[0] + accp[1]
    return jnp.dot(s, w2[...], preferred_element_type=F32), b2[0:1, 0:LAT]


def _node0_kernel(accp_ref, cntp_ref, xw_ref, w2_ref, b2_ref, w1s_ref,
                  w1d_ref, b1_ref, a_ref, b_ref, cnt0_ref, cntc_ref):
    sw2, b2 = _node_common(accp_ref[...], w2_ref, b2_ref)
    cnt0 = (cntp_ref[0, :, 0] + cntp_ref[1, :, 0]).reshape(-1, 1)
    cntc = jnp.maximum(cnt0, 1.0)
    agg = (sw2 + cnt0 * b2) / cntc
    xx = jnp.maximum(agg, 0.0)
    a_ref[...] = (jnp.dot(xx, w1s_ref[0:LAT, :], preferred_element_type=F32)
                  + xw_ref[:, 16:32] + b1_ref[0:1, :])
    b_ref[...] = jnp.dot(xx, w1d_ref[0:LAT, :], preferred_element_type=F32)
    cnt0_ref[...] = cnt0
    cntc_ref[...] = cntc


def _node_mid_kernel(i, accp_ref, cnt0_ref, cntc_ref, xw_ref, w2_ref, b2_ref,
                     w1s_ref, w1d_ref, b1_ref, a_ref, b_ref):
    sw2, b2 = _node_common(accp_ref[...], w2_ref, b2_ref)
    agg = (sw2 + cnt0_ref[...] * b2) / cntc_ref[...]
    xx = jnp.maximum(agg, 0.0)
    a_ref[...] = (jnp.dot(xx, w1s_ref[0:LAT, :], preferred_element_type=F32)
                  + xw_ref[:, 16 * (i + 1):16 * (i + 2)] + b1_ref[0:1, :])
    b_ref[...] = jnp.dot(xx, w1d_ref[0:LAT, :], preferred_element_type=F32)


def _node_last_kernel(accp_ref, cnt0_ref, cntc_ref, w2_ref, b2_ref, wf_ref,
                      bf_ref, xlast_ref, p_ref, pm_ref):
    sw2, b2 = _node_common(accp_ref[...], w2_ref, b2_ref)
    agg = (sw2 + cnt0_ref[...] * b2) / cntc_ref[...]
    xx = jnp.maximum(agg, 0.0)
    p = jnp.maximum(
        jnp.dot(xx, wf_ref[0:LAT, :], preferred_element_type=F32)
        + bf_ref[0:1, :], 0.0)
    p_ref[...] = p
    xl = xlast_ref[...]
    pm_ref[...] = jnp.where(xl != 0.0, xl, p)


def _node0_call(accp, cntp, xw, w2, b2, w1s, w1d, b1):
    full = lambda shape: pl.BlockSpec(shape, lambda i: tuple(0 for _ in shape))
    return pl.pallas_call(
        _node0_kernel,
        grid=(_NGRID,),
        in_specs=[
            pl.BlockSpec((NC, _NBLK, 16), lambda i: (0, i, 0)),
            pl.BlockSpec((NC, _NBLK, 16), lambda i: (0, i, 0)),
            pl.BlockSpec((_NBLK, DN), lambda i: (i, 0)),
            full((16, LAT)), full((8, 8)), full((8, 16)), full((8, 16)),
            full((8, 16)),
        ],
        out_specs=[
            pl.BlockSpec((_NBLK, 16), lambda i: (i, 0)),
            pl.BlockSpec((_NBLK, 16), lambda i: (i, 0)),
            pl.BlockSpec((_NBLK, 1), lambda i: (i, 0)),
            pl.BlockSpec((_NBLK, 1), lambda i: (i, 0)),
        ],
        out_shape=[
            jax.ShapeDtypeStruct((NPAD, 16), F32),
            jax.ShapeDtypeStruct((NPAD, 16), F32),
            jax.ShapeDtypeStruct((NPAD, 1), F32),
            jax.ShapeDtypeStruct((NPAD, 1), F32),
        ],
    )(accp, cntp, xw, w2, b2, w1s, w1d, b1)


def _node_mid_call(i, accp, cnt0, cntc, xw, w2, b2, w1s, w1d, b1):
    full = lambda shape: pl.BlockSpec(shape, lambda i_: tuple(0 for _ in shape))
    return pl.pallas_call(
        functools.partial(_node_mid_kernel, i),
        grid=(_NGRID,),
        in_specs=[
            pl.BlockSpec((NC, _NBLK, 16), lambda i_: (0, i_, 0)),
            pl.BlockSpec((_NBLK, 1), lambda i_: (i_, 0)),
            pl.BlockSpec((_NBLK, 1), lambda i_: (i_, 0)),
            pl.BlockSpec((_NBLK, DN), lambda i_: (i_, 0)),
            full((16, LAT)), full((8, 8)), full((8, 16)), full((8, 16)),
            full((8, 16)),
        ],
        out_specs=[
            pl.BlockSpec((_NBLK, 16), lambda i_: (i_, 0)),
            pl.BlockSpec((_NBLK, 16), lambda i_: (i_, 0)),
        ],
        out_shape=[
            jax.ShapeDtypeStruct((NPAD, 16), F32),
            jax.ShapeDtypeStruct((NPAD, 16), F32),
        ],
    )(accp, cnt0, cntc, xw, w2, b2, w1s, w1d, b1)


def _node_last_call(accp, cnt0, cntc, w2, b2, wf, bf, xlast):
    full = lambda shape: pl.BlockSpec(shape, lambda i: tuple(0 for _ in shape))
    return pl.pallas_call(
        _node_last_kernel,
        grid=(_NGRID,),
        in_specs=[
            pl.BlockSpec((NC, _NBLK, 16), lambda i: (0, i, 0)),
            pl.BlockSpec((_NBLK, 1), lambda i: (i, 0)),
            pl.BlockSpec((_NBLK, 1), lambda i: (i, 0)),
            full((16, LAT)), full((8, 8)), full((8, 1)), full((8, 1)),
            pl.BlockSpec((_NBLK, 1), lambda i: (i, 0)),
        ],
        out_specs=[
            pl.BlockSpec((_NBLK, 1), lambda i: (i, 0)),
            pl.BlockSpec((_NBLK, 1), lambda i: (i, 0)),
        ],
        out_shape=[
            jax.ShapeDtypeStruct((NPAD, 1), F32),
            jax.ShapeDtypeStruct((NPAD, 1), F32),
        ],
    )(accp, cnt0, cntc, w2, b2, wf, bf, xlast)


def _imbal_kernel(netp_ref, p_ref, out_ref):
    net = p_ref[0:1, :] + jnp.sum(netp_ref[...], axis=0, keepdims=True)
    out_ref[...] = jnp.sum(jnp.abs(net)).reshape(1, 1)


def _imbal_call(netp, p_row):
    return pl.pallas_call(
        _imbal_kernel,
        grid=(1,),
        in_specs=[
            pl.BlockSpec((NW, NPAD), lambda i: (0, 0)),
            pl.BlockSpec((8, NPAD), lambda i: (0, 0)),
        ],
        out_specs=pl.BlockSpec((1, 1), lambda i: (0, 0)),
        out_shape=jax.ShapeDtypeStruct((1, 1), F32),
    )(netp, p_row)


def kernel(x, edge_index, edge_attr, W1, b1, W2, b2, Wf, bf):
    src = edge_index[0]
    dst = edge_index[1]
    npad_e = E_PAD - E
    src_p = jnp.concatenate([src, jnp.zeros((npad_e,), jnp.int32)])
    dst_p = jnp.concatenate([dst, jnp.full((npad_e,), N, jnp.int32)])
    ea_p = jnp.concatenate([edge_attr, jnp.zeros((npad_e, DE), F32)], axis=0)
    x_pad = jnp.concatenate([x, jnp.zeros((NPAD - N, DN), F32)], axis=0)
    xlast_pad = x_pad[:, DN - 1:DN]

    w1s = W1[:, 0:LAT, :]
    w1d = W1[:, LAT:2 * LAT, :]
    w1x = W1[:, 2 * LAT:2 * LAT + DN, :]
    w1e = W1[:, 2 * LAT + DN:, :]
    wx_all = jnp.transpose(w1x, (1, 0, 2)).reshape(DN, NCONV * 16)
    we_all = jnp.concatenate([
        jnp.transpose(w1e, (1, 0, 2)).reshape(DE, NCONV * 16),
        jnp.zeros((8 - DE, NCONV * 16), F32)], axis=0)
    w1s_p = jnp.concatenate([w1s, jnp.zeros((NCONV, 2, 16), F32)], axis=1)
    w1d_p = jnp.concatenate([w1d, jnp.zeros((NCONV, 2, 16), F32)], axis=1)
    b1_bc = jnp.broadcast_to(b1[:, None, :], (NCONV, 8, 16))
    b2_bc = jnp.broadcast_to(
        jnp.pad(b2, ((0, 0), (0, 2)))[:, None, :], (NCONV, 8, 8))
    wf_p = jnp.concatenate([Wf, jnp.zeros((2, 1), F32)], axis=0)
    bf_bc = jnp.broadcast_to(bf.reshape(1, 1), (8, 1))

    xw, a0 = _prep_call(x_pad, wx_all, b1_bc[0])
    ews = _ew_call(ea_p, we_all)

    edge0 = _make_edge0()
    edge = _make_edge()

    accp, cntp = edge0(a0, ews[0], src_p, dst_p)
    a_t, b_t, cnt0, cntc = _node0_call(
        accp, cntp, xw, W2[0], b2_bc[0], w1s_p[1], w1d_p[1], b1_bc[1])

    for i in range(1, NCONV - 1):
        accp = edge(a_t, b_t, ews[i], src_p, dst_p)
        a_t, b_t = _node_mid_call(
            i, accp, cnt0, cntc, xw, W2[i], b2_bc[i],
            w1s_p[i + 1], w1d_p[i + 1], b1_bc[i + 1])

    accp = edge(a_t, b_t, ews[NCONV - 1], src_p, dst_p)
    p_full, pm_full = _node_last_call(
        accp, cnt0, cntc, W2[NCONV - 1], b2_bc[NCONV - 1], wf_p, bf_bc,
        xlast_pad)

    p_tab = jnp.concatenate([pm_full[:N, 0], jnp.zeros((NPAD - N,), F32)])
    ea0_p = ea_p[:, 0]

    balance = _make_balance()
    flow_pad, netp = balance(p_tab, src_p, dst_p, ea0_p)

    p_row = jnp.broadcast_to(p_tab[None, :], (8, NPAD))
    imb = _imbal_call(netp, p_row)

    P = p_full[:N]
    flow = flow_pad[:E].reshape(E, 1)
    return (P, flow, imb.reshape(1))


# speedup vs baseline: 10.6173x; 1.2775x over previous
import functools

import jax
import jax.numpy as jnp
from jax import lax
from jax.experimental import pallas as pl
from jax.experimental.pallas import tpu as pltpu
from jax.experimental.pallas import tpu_sc as plsc

N = 10000
E = 320000
DN = 128
DE = 4
LAT = 6
H = 16
NCONV = 8

NC = 2
NS = 16
NW = NC * NS
NPAD = N + 112
SUP = 1024
CHB = SUP // 128
EPW = 10240
E_PAD = EPW * NW
NSUP = EPW // SUP
RPT = NPAD // NS

F32 = jnp.float32


def _mesh():
    return plsc.VectorSubcoreMesh(
        core_axis_name="c", subcore_axis_name="s", num_cores=NC, num_subcores=NS
    )


def _edge_body_common(a_hbm, b_hbm, ew_hbm, src_hbm, dst2_hbm, accp_hbm,
                      cntp_hbm, sidx, didx, abufs, bbufs, ewbufs, obuf,
                      acc_sh, cnt_sh, sem_a, sem_b, sem_m, with_b, with_cnt):
    c = lax.axis_index("c")
    s = lax.axis_index("s")
    wid = c * NS + s
    rpg = EPW // 128

    pltpu.sync_copy(src_hbm.at[pl.ds(wid * EPW, EPW)], sidx)
    pltpu.sync_copy(dst2_hbm.at[pl.ds(wid * rpg, rpg)], didx)

    zero16 = jnp.zeros((16,), F32)
    az = abufs[0]

    @plsc.parallel_loop(0, RPT, 1, unroll=8)
    def zrow(r):
        az[r] = zero16

    pltpu.sync_copy(az.at[pl.ds(0, RPT)], acc_sh.at[pl.ds(s * RPT, RPT)])
    if with_cnt:
        lane = lax.iota(jnp.int32, 16)
        onerow = jnp.where(lane == 0, 1.0, 0.0).astype(F32)

        @plsc.parallel_loop(0, 128, 1, unroll=8)
        def orow(r):
            obuf[r] = onerow

        pltpu.sync_copy(az.at[pl.ds(0, RPT)], cnt_sh.at[pl.ds(s * RPT, RPT)])
    plsc.subcore_barrier()

    base0 = wid * EPW

    def fill(g, bi):
        base = base0 + g * SUP
        pltpu.async_copy(ew_hbm.at[pl.ds(base, SUP)], ewbufs[bi], sem_m)
        for j in range(CHB):
            pltpu.async_copy(
                a_hbm.at[sidx.at[pl.ds(g * SUP + j * 128, 128)]],
                abufs[bi].at[pl.ds(j * 128, 128)], sem_a)
        if with_b:
            for j in range(CHB):
                pltpu.async_copy(
                    b_hbm.at[didx.at[g * CHB + j]],
                    bbufs[bi].at[pl.ds(j * 128, 128)], sem_b)

    def process(g, bi):
        pltpu.make_async_copy(ew_hbm.at[pl.ds(0, SUP)], ewbufs[bi],
                              sem_m).wait()
        pltpu.make_async_copy(a_hbm.at[pl.ds(0, SUP)], abufs[bi],
                              sem_a).wait()
        ab = abufs[bi]
        eb = ewbufs[bi]
        if with_b:
            pltpu.make_async_copy(a_hbm.at[pl.ds(0, SUP)], bbufs[bi],
                                  sem_b).wait()
            bb = bbufs[bi]

            @plsc.parallel_loop(0, SUP, 1, unroll=8)
            def comp(e):
                ab[e] = jnp.maximum(ab[e] + bb[e] + eb[e], 0.0)
        else:
            @plsc.parallel_loop(0, SUP, 1, unroll=8)
            def comp(e):
                ab[e] = jnp.maximum(ab[e] + eb[e], 0.0)

        for j in range(CHB):
            pltpu.sync_copy(ab.at[pl.ds(j * 128, 128)],
                            acc_sh.at[didx.at[g * CHB + j]], add=True)
        if with_cnt:
            for j in range(CHB):
                pltpu.sync_copy(obuf, cnt_sh.at[didx.at[g * CHB + j]],
                                add=True)

    fill(0, 0)
    fill(1, 1)

    def lbody(i, carry):
        g0 = 2 * i
        process(g0, 0)
        fill(g0 + 2, 0)
        process(g0 + 1, 1)
        fill(g0 + 3, 1)
        return carry

    lax.fori_loop(0, NSUP // 2 - 1, lbody, 0)
    process(NSUP - 2, 0)
    process(NSUP - 1, 1)

    plsc.subcore_barrier()
    pltpu.sync_copy(acc_sh.at[pl.ds(s * RPT, RPT)],
                    accp_hbm.at[c, pl.ds(s * RPT, RPT)])
    if with_cnt:
        pltpu.sync_copy(cnt_sh.at[pl.ds(s * RPT, RPT)],
                        cntp_hbm.at[c, pl.ds(s * RPT, RPT)])


def _make_edge0():
    out_type = (
        jax.ShapeDtypeStruct((NC, NPAD, 16), F32),
        jax.ShapeDtypeStruct((NC, NPAD, 16), F32),
    )
    scratch = [
        pltpu.VMEM((EPW,), jnp.int32),
        pltpu.VMEM((EPW // 128, 128), jnp.int32),
        pltpu.VMEM((SUP, 16), F32),
        pltpu.VMEM((SUP, 16), F32),
        pltpu.VMEM((SUP, 16), F32),
        pltpu.VMEM((SUP, 16), F32),
        pltpu.VMEM((128, 16), F32),
        pltpu.VMEM_SHARED((NPAD, 16), F32),
        pltpu.VMEM_SHARED((NPAD, 16), F32),
        pltpu.SemaphoreType.DMA,
        pltpu.SemaphoreType.DMA,
    ]

    @functools.partial(pl.kernel, out_type=out_type, mesh=_mesh(),
                       scratch_types=scratch,
                       compiler_params=pltpu.CompilerParams(
                           use_tc_tiling_on_sc=False, needs_layout_passes=False))
    def k(a_hbm, ew_hbm, src_hbm, dst2_hbm, accp_hbm, cntp_hbm,
          sidx, didx, abuf0, abuf1, ewbuf0, ewbuf1, obuf, acc_sh, cnt_sh,
          sem_a, sem_m):
        _edge_body_common(a_hbm, None, ew_hbm, src_hbm, dst2_hbm, accp_hbm,
                          cntp_hbm, sidx, didx, (abuf0, abuf1), None,
                          (ewbuf0, ewbuf1), obuf, acc_sh, cnt_sh,
                          sem_a, None, sem_m, with_b=False, with_cnt=True)

    return k


def _make_edge():
    out_type = jax.ShapeDtypeStruct((NC, NPAD, 16), F32)
    scratch = [
        pltpu.VMEM((EPW,), jnp.int32),
        pltpu.VMEM((EPW // 128, 128), jnp.int32),
        pltpu.VMEM((SUP, 16), F32),
        pltpu.VMEM((SUP, 16), F32),
        pltpu.VMEM((SUP, 16), F32),
        pltpu.VMEM((SUP, 16), F32),
        pltpu.VMEM((SUP, 16), F32),
        pltpu.VMEM((SUP, 16), F32),
        pltpu.VMEM_SHARED((NPAD, 16), F32),
        pltpu.SemaphoreType.DMA,
        pltpu.SemaphoreType.DMA,
        pltpu.SemaphoreType.DMA,
    ]

    @functools.partial(pl.kernel, out_type=out_type, mesh=_mesh(),
                       scratch_types=scratch,
                       compiler_params=pltpu.CompilerParams(
                           use_tc_tiling_on_sc=False, needs_layout_passes=False))
    def k(a_hbm, b_hbm, ew_hbm, src_hbm, dst2_hbm, accp_hbm,
          sidx, didx, abuf0, abuf1, bbuf0, bbuf1, ewbuf0, ewbuf1, acc_sh,
          sem_a, sem_b, sem_m):
        _edge_body_common(a_hbm, b_hbm, ew_hbm, src_hbm, dst2_hbm, accp_hbm,
                          None, sidx, didx, (abuf0, abuf1), (bbuf0, bbuf1),
                          (ewbuf0, ewbuf1), None, acc_sh, None,
                          sem_a, sem_b, sem_m, with_b=True, with_cnt=False)

    return k


def _make_balance():
    out_type = (
        jax.ShapeDtypeStruct((E_PAD,), F32),
        jax.ShapeDtypeStruct((NW, NPAD), F32),
    )
    scratch = [
        pltpu.VMEM((NPAD,), F32),
        pltpu.VMEM((NPAD,), F32),
        pltpu.VMEM((EPW,), jnp.int32),
        pltpu.VMEM((EPW,), jnp.int32),
        pltpu.VMEM((EPW,), F32),
        pltpu.VMEM((EPW,), F32),
    ]

    @functools.partial(pl.kernel, out_type=out_type, mesh=_mesh(),
                       scratch_types=scratch,
                       compiler_params=pltpu.CompilerParams(
                           use_tc_tiling_on_sc=False, needs_layout_passes=False))
    def k(p_hbm, src_hbm, dst_hbm, ea_hbm, flow_hbm, netp_hbm,
          ptab, netacc, sbuf, dbuf, eabuf, fbuf):
        c = lax.axis_index("c")
        s = lax.axis_index("s")
        wid = c * NS + s
        base0 = wid * EPW
        pltpu.sync_copy(p_hbm, ptab)
        pltpu.sync_copy(src_hbm.at[pl.ds(base0, EPW)], sbuf)
        pltpu.sync_copy(dst_hbm.at[pl.ds(base0, EPW)], dbuf)
        pltpu.sync_copy(ea_hbm.at[pl.ds(base0, EPW)], eabuf)

        zero16 = jnp.zeros((16,), F32)

        @plsc.parallel_loop(0, NPAD // 16, 1, unroll=8)
        def zr(r):
            netacc[pl.ds(r * 16, 16)] = zero16

        lane = lax.iota(jnp.int32, 16)

        def v16(kk, carry):
            sv = sbuf[pl.ds(kk * 16, 16)]
            dv = dbuf[pl.ds(kk * 16, 16)]
            ps = plsc.load_gather(ptab, [sv])
            pd = plsc.load_gather(ptab, [dv])
            fl = (ps - pd) * eabuf[pl.ds(kk * 16, 16)]
            fbuf[pl.ds(kk * 16, 16)] = fl
            nfl = -fl
            for j in range(16):
                m = lane == j
                plsc.addupdate_scatter(netacc, [sv], nfl, mask=m)
                plsc.addupdate_scatter(netacc, [dv], fl, mask=m)
            return carry

        lax.fori_loop(0, EPW // 16, v16, 0, unroll=2)
        pltpu.sync_copy(fbuf, flow_hbm.at[pl.ds(base0, EPW)])
        pltpu.sync_copy(netacc, netp_hbm.at[wid])

    return k


_NBLK = NPAD // 4
_NGRID = NPAD // _NBLK
_EBLK = 4096
_EGRID = E_PAD // _EBLK


def _prep_kernel(x_ref, wx_ref, b1_ref, xw_ref, a0_ref):
    xw = jnp.dot(x_ref[...], wx_ref[...], preferred_element_type=F32)
    xw_ref[...] = xw
    a0_ref[...] = xw[:, 0:16] + b1_ref[0:1, :]


def _prep_call(x_pad, wx, b1_0):
    return pl.pallas_call(
        _prep_kernel,
        grid=(_NGRID,),
        in_specs=[
            pl.BlockSpec((_NBLK, DN), lambda i: (i, 0)),
            pl.BlockSpec((DN, DN), lambda i: (0, 0)),
            pl.BlockSpec((8, 16), lambda i: (0, 0)),
        ],
        out_specs=[
            pl.BlockSpec((_NBLK, DN), lambda i: (i, 0)),
            pl.BlockSpec((_NBLK, 16), lambda i: (i, 0)),
        ],
        out_shape=[
            jax.ShapeDtypeStruct((NPAD, DN), F32),
            jax.ShapeDtypeStruct((NPAD, 16), F32),
        ],
    )(x_pad, wx, b1_0)


def _ew_kernel(ea_ref, we_ref, *out_refs):
    r = jnp.dot(ea_ref[...], we_ref[0:DE, :], preferred_element_type=F32)
    for i, o in enumerate(out_refs):
        o[...] = r[:, 16 * i:16 * (i + 1)]


def _ew_call(ea_pad, we):
    return pl.pallas_call(
        _ew_kernel,
        grid=(_EGRID,),
        in_specs=[
            pl.BlockSpec((_EBLK, DE), lambda i: (i, 0)),
            pl.BlockSpec((8, DN), lambda i: (0, 0)),
        ],
        out_specs=[pl.BlockSpec((_EBLK, 16), lambda i: (i, 0))
                   for _ in range(NCONV)],
        out_shape=[jax.ShapeDtypeStruct((E_PAD, 16), F32)
                   for _ in range(NCONV)],
    )(ea_pad, we)


def _node_common(accp, w2, b2):
    s = accp[0] + accp[1]
    return jnp.dot(s, w2[...], preferred_element_type=F32), b2[0:1, 0:LAT]


def _node0_kernel(accp_ref, cntp_ref, xw_ref, w2_ref, b2_ref, w1s_ref,
                  w1d_ref, b1_ref, a_ref, b_ref, cnt0_ref, cntc_ref):
    sw2, b2 = _node_common(accp_ref[...], w2_ref, b2_ref)
    cnt0 = (cntp_ref[0, :, 0] + cntp_ref[1, :, 0]).reshape(-1, 1)
    cntc = jnp.maximum(cnt0, 1.0)
    agg = (sw2 + cnt0 * b2) / cntc
    xx = jnp.maximum(agg, 0.0)
    a_ref[...] = (jnp.dot(xx, w1s_ref[0:LAT, :], preferred_element_type=F32)
                  + xw_ref[:, 16:32] + b1_ref[0:1, :])
    b_ref[...] = jnp.dot(xx, w1d_ref[0:LAT, :], preferred_element_type=F32)
    cnt0_ref[...] = cnt0
    cntc_ref[...] = cntc


def _node_mid_kernel(i, accp_ref, cnt0_ref, cntc_ref, xw_ref, w2_ref, b2_ref,
                     w1s_ref, w1d_ref, b1_ref, a_ref, b_ref):
    sw2, b2 = _node_common(accp_ref[...], w2_ref, b2_ref)
    agg = (sw2 + cnt0_ref[...] * b2) / cntc_ref[...]
    xx = jnp.maximum(agg, 0.0)
    a_ref[...] = (jnp.dot(xx, w1s_ref[0:LAT, :], preferred_element_type=F32)
                  + xw_ref[:, 16 * (i + 1):16 * (i + 2)] + b1_ref[0:1, :])
    b_ref[...] = jnp.dot(xx, w1d_ref[0:LAT, :], preferred_element_type=F32)


def _node_last_kernel(accp_ref, cnt0_ref, cntc_ref, w2_ref, b2_ref, wf_ref,
                      bf_ref, xlast_ref, p_ref, pm_ref):
    sw2, b2 = _node_common(accp_ref[...], w2_ref, b2_ref)
    agg = (sw2 + cnt0_ref[...] * b2) / cntc_ref[...]
    xx = jnp.maximum(agg, 0.0)
    p = jnp.maximum(
        jnp.dot(xx, wf_ref[0:LAT, :], preferred_element_type=F32)
        + bf_ref[0:1, :], 0.0)
    p_ref[...] = p
    xl = xlast_ref[...]
    pm_ref[...] = jnp.where(xl != 0.0, xl, p)


def _node0_call(accp, cntp, xw, w2, b2, w1s, w1d, b1):
    full = lambda shape: pl.BlockSpec(shape, lambda i: tuple(0 for _ in shape))
    return pl.pallas_call(
        _node0_kernel,
        grid=(_NGRID,),
        in_specs=[
            pl.BlockSpec((NC, _NBLK, 16), lambda i: (0, i, 0)),
            pl.BlockSpec((NC, _NBLK, 16), lambda i: (0, i, 0)),
            pl.BlockSpec((_NBLK, DN), lambda i: (i, 0)),
            full((16, LAT)), full((8, 8)), full((8, 16)), full((8, 16)),
            full((8, 16)),
        ],
        out_specs=[
            pl.BlockSpec((_NBLK, 16), lambda i: (i, 0)),
            pl.BlockSpec((_NBLK, 16), lambda i: (i, 0)),
            pl.BlockSpec((_NBLK, 1), lambda i: (i, 0)),
            pl.BlockSpec((_NBLK, 1), lambda i: (i, 0)),
        ],
        out_shape=[
            jax.ShapeDtypeStruct((NPAD, 16), F32),
            jax.ShapeDtypeStruct((NPAD, 16), F32),
            jax.ShapeDtypeStruct((NPAD, 1), F32),
            jax.ShapeDtypeStruct((NPAD, 1), F32),
        ],
    )(accp, cntp, xw, w2, b2, w1s, w1d, b1)


def _node_mid_call(i, accp, cnt0, cntc, xw, w2, b2, w1s, w1d, b1):
    full = lambda shape: pl.BlockSpec(shape, lambda i_: tuple(0 for _ in shape))
    return pl.pallas_call(
        functools.partial(_node_mid_kernel, i),
        grid=(_NGRID,),
        in_specs=[
            pl.BlockSpec((NC, _NBLK, 16), lambda i_: (0, i_, 0)),
            pl.BlockSpec((_NBLK, 1), lambda i_: (i_, 0)),
            pl.BlockSpec((_NBLK, 1), lambda i_: (i_, 0)),
            pl.BlockSpec((_NBLK, DN), lambda i_: (i_, 0)),
            full((16, LAT)), full((8, 8)), full((8, 16)), full((8, 16)),
            full((8, 16)),
        ],
        out_specs=[
            pl.BlockSpec((_NBLK, 16), lambda i_: (i_, 0)),
            pl.BlockSpec((_NBLK, 16), lambda i_: (i_, 0)),
        ],
        out_shape=[
            jax.ShapeDtypeStruct((NPAD, 16), F32),
            jax.ShapeDtypeStruct((NPAD, 16), F32),
        ],
    )(accp, cnt0, cntc, xw, w2, b2, w1s, w1d, b1)


def _node_last_call(accp, cnt0, cntc, w2, b2, wf, bf, xlast):
    full = lambda shape: pl.BlockSpec(shape, lambda i: tuple(0 for _ in shape))
    return pl.pallas_call(
        _node_last_kernel,
        grid=(_NGRID,),
        in_specs=[
            pl.BlockSpec((NC, _NBLK, 16), lambda i: (0, i, 0)),
            pl.BlockSpec((_NBLK, 1), lambda i: (i, 0)),
            pl.BlockSpec((_NBLK, 1), lambda i: (i, 0)),
            full((16, LAT)), full((8, 8)), full((8, 1)), full((8, 1)),
            pl.BlockSpec((_NBLK, 1), lambda i: (i, 0)),
        ],
        out_specs=[
            pl.BlockSpec((_NBLK, 1), lambda i: (i, 0)),
            pl.BlockSpec((_NBLK, 1), lambda i: (i, 0)),
        ],
        out_shape=[
            jax.ShapeDtypeStruct((NPAD, 1), F32),
            jax.ShapeDtypeStruct((NPAD, 1), F32),
        ],
    )(accp, cnt0, cntc, w2, b2, wf, bf, xlast)


def _imbal_kernel(netp_ref, p_ref, out_ref):
    net = p_ref[0:1, :] + jnp.sum(netp_ref[...], axis=0, keepdims=True)
    out_ref[...] = jnp.sum(jnp.abs(net)).reshape(1, 1)


def _imbal_call(netp, p_row):
    return pl.pallas_call(
        _imbal_kernel,
        grid=(1,),
        in_specs=[
            pl.BlockSpec((NW, NPAD), lambda i: (0, 0)),
            pl.BlockSpec((8, NPAD), lambda i: (0, 0)),
        ],
        out_specs=pl.BlockSpec((1, 1), lambda i: (0, 0)),
        out_shape=jax.ShapeDtypeStruct((1, 1), F32),
    )(netp, p_row)


def kernel(x, edge_index, edge_attr, W1, b1, W2, b2, Wf, bf):
    src = edge_index[0]
    dst = edge_index[1]
    npad_e = E_PAD - E
    src_p = jnp.concatenate([src, jnp.zeros((npad_e,), jnp.int32)])
    dst_p = jnp.concatenate([dst, jnp.full((npad_e,), N, jnp.int32)])
    dst2_p = dst_p.reshape(E_PAD // 128, 128)
    ea_p = jnp.concatenate([edge_attr, jnp.zeros((npad_e, DE), F32)], axis=0)
    x_pad = jnp.concatenate([x, jnp.zeros((NPAD - N, DN), F32)], axis=0)
    xlast_pad = x_pad[:, DN - 1:DN]

    w1s = W1[:, 0:LAT, :]
    w1d = W1[:, LAT:2 * LAT, :]
    w1x = W1[:, 2 * LAT:2 * LAT + DN, :]
    w1e = W1[:, 2 * LAT + DN:, :]
    wx_all = jnp.transpose(w1x, (1, 0, 2)).reshape(DN, NCONV * 16)
    we_all = jnp.concatenate([
        jnp.transpose(w1e, (1, 0, 2)).reshape(DE, NCONV * 16),
        jnp.zeros((8 - DE, NCONV * 16), F32)], axis=0)
    w1s_p = jnp.concatenate([w1s, jnp.zeros((NCONV, 2, 16), F32)], axis=1)
    w1d_p = jnp.concatenate([w1d, jnp.zeros((NCONV, 2, 16), F32)], axis=1)
    b1_bc = jnp.broadcast_to(b1[:, None, :], (NCONV, 8, 16))
    b2_bc = jnp.broadcast_to(
        jnp.pad(b2, ((0, 0), (0, 2)))[:, None, :], (NCONV, 8, 8))
    wf_p = jnp.concatenate([Wf, jnp.zeros((2, 1), F32)], axis=0)
    bf_bc = jnp.broadcast_to(bf.reshape(1, 1), (8, 1))

    xw, a0 = _prep_call(x_pad, wx_all, b1_bc[0])
    ews = _ew_call(ea_p, we_all)

    edge0 = _make_edge0()
    edge = _make_edge()

    accp, cntp = edge0(a0, ews[0], src_p, dst2_p)
    a_t, b_t, cnt0, cntc = _node0_call(
        accp, cntp, xw, W2[0], b2_bc[0], w1s_p[1], w1d_p[1], b1_bc[1])

    for i in range(1, NCONV - 1):
        accp = edge(a_t, b_t, ews[i], src_p, dst2_p)
        a_t, b_t = _node_mid_call(
            i, accp, cnt0, cntc, xw, W2[i], b2_bc[i],
            w1s_p[i + 1], w1d_p[i + 1], b1_bc[i + 1])

    accp = edge(a_t, b_t, ews[NCONV - 1], src_p, dst2_p)
    p_full, pm_full = _node_last_call(
        accp, cnt0, cntc, W2[NCONV - 1], b2_bc[NCONV - 1], wf_p, bf_bc,
        xlast_pad)

    p_tab = jnp.concatenate([pm_full[:N, 0], jnp.zeros((NPAD - N,), F32)])
    ea0_p = ea_p[:, 0]

    balance = _make_balance()
    flow_pad, netp = balance(p_tab, src_p, dst_p, ea0_p)

    p_row = jnp.broadcast_to(p_tab[None, :], (8, NPAD))
    imb = _imbal_call(netp, p_row)

    P = p_full[:N]
    flow = flow_pad[:E].reshape(E, 1)
    return (P, flow, imb.reshape(1))


# speedup vs baseline: 15.8587x; 1.4937x over previous
import functools

import jax
import jax.numpy as jnp
from jax import lax
from jax.experimental import pallas as pl
from jax.experimental.pallas import tpu as pltpu
from jax.experimental.pallas import tpu_sc as plsc

N = 10000
E = 320000
DN = 128
DE = 4
LAT = 6
H = 16
NCONV = 8

NC = 2
NS = 16
NW = NC * NS
NPAD = N + 112
SUP = 1024
CHB = SUP // 128
EPW = 10240
E_PAD = EPW * NW
NSUP = EPW // SUP
RPT = NPAD // NS

F32 = jnp.float32


def _mesh():
    return plsc.VectorSubcoreMesh(
        core_axis_name="c", subcore_axis_name="s", num_cores=NC, num_subcores=NS
    )


def _edge_body_common(a_hbm, b_hbm, ew_hbm, src3_hbm, dst3_hbm, accp_hbm,
                      cntp_hbm, sidx, didx, abufs, bbufs, ewbufs, obuf,
                      acc_sh, cnt_sh, sem_a, sem_b, sem_m, with_b, with_cnt):
    c = lax.axis_index("c")
    s = lax.axis_index("s")
    wid = c * NS + s

    pltpu.sync_copy(src3_hbm.at[pl.ds(wid * NSUP, NSUP)], sidx)
    pltpu.sync_copy(dst3_hbm.at[pl.ds(wid * NSUP, NSUP)], didx)

    zero16 = jnp.zeros((16,), F32)
    az = abufs[0]

    @plsc.parallel_loop(0, RPT, 1, unroll=8)
    def zrow(r):
        az[r] = zero16

    pltpu.sync_copy(az.at[pl.ds(0, RPT)], acc_sh.at[pl.ds(s * RPT, RPT)])
    if with_cnt:
        lane = lax.iota(jnp.int32, 16)
        onerow = jnp.where(lane == 0, 1.0, 0.0).astype(F32)

        @plsc.parallel_loop(0, SUP, 1, unroll=8)
        def orow(r):
            obuf[r] = onerow

        pltpu.sync_copy(az.at[pl.ds(0, RPT)], cnt_sh.at[pl.ds(s * RPT, RPT)])
    plsc.subcore_barrier()

    base0 = wid * EPW

    def fill(g, bi):
        base = base0 + g * SUP
        pltpu.async_copy(ew_hbm.at[pl.ds(base // 8, SUP // 8)], ewbufs[bi],
                         sem_m)
        for j in range(CHB):
            pltpu.async_copy(a_hbm.at[sidx.at[g, j]],
                             abufs[bi].at[pl.ds(j * 128, 128)], sem_a)
        if with_b:
            for j in range(CHB):
                pltpu.async_copy(b_hbm.at[didx.at[g, j]],
                                 bbufs[bi].at[pl.ds(j * 128, 128)], sem_b)

    def process(g, bi):
        pltpu.make_async_copy(ew_hbm.at[pl.ds(0, SUP // 8)], ewbufs[bi],
                              sem_m).wait()
        pltpu.make_async_copy(a_hbm.at[pl.ds(0, SUP)], abufs[bi],
                              sem_a).wait()
        ab = abufs[bi]
        eb = ewbufs[bi]
        if with_b:
            pltpu.make_async_copy(a_hbm.at[pl.ds(0, SUP)], bbufs[bi],
                                  sem_b).wait()
            bb = bbufs[bi]

            @plsc.parallel_loop(0, SUP // 8, 1, unroll=2)
            def comp(e8):
                for j in range(8):
                    e = e8 * 8 + j
                    ab[e] = jnp.maximum(
                        ab[e] + bb[e] + eb[e8, pl.ds(j * 16, 16)], 0.0)
        else:
            @plsc.parallel_loop(0, SUP // 8, 1, unroll=2)
            def comp(e8):
                for j in range(8):
                    e = e8 * 8 + j
                    ab[e] = jnp.maximum(
                        ab[e] + eb[e8, pl.ds(j * 16, 16)], 0.0)

        for j in range(CHB):
            pltpu.sync_copy(ab.at[pl.ds(j * 128, 128)],
                            acc_sh.at[didx.at[g, j]], add=True)
        if with_cnt:
            for j in range(CHB):
                pltpu.sync_copy(obuf.at[pl.ds(j * 128, 128)],
                                cnt_sh.at[didx.at[g, j]], add=True)

    fill(0, 0)
    fill(1, 1)

    def lbody(i, carry):
        g0 = 2 * i
        process(g0, 0)
        fill(g0 + 2, 0)
        process(g0 + 1, 1)
        fill(g0 + 3, 1)
        return carry

    lax.fori_loop(0, NSUP // 2 - 1, lbody, 0)
    process(NSUP - 2, 0)
    process(NSUP - 1, 1)

    plsc.subcore_barrier()
    pltpu.sync_copy(acc_sh.at[pl.ds(s * RPT, RPT)],
                    accp_hbm.at[c, pl.ds(s * RPT, RPT)])
    if with_cnt:
        pltpu.sync_copy(cnt_sh.at[pl.ds(s * RPT, RPT)],
                        cntp_hbm.at[c, pl.ds(s * RPT, RPT)])


def _make_edge0():
    out_type = (
        jax.ShapeDtypeStruct((NC, NPAD, 16), F32),
        jax.ShapeDtypeStruct((NC, NPAD, 16), F32),
    )
    scratch = [
        pltpu.VMEM((NSUP, CHB, 128), jnp.int32),
        pltpu.VMEM((NSUP, CHB, 128), jnp.int32),
        pltpu.VMEM((SUP, 16), F32),
        pltpu.VMEM((SUP, 16), F32),
        pltpu.VMEM((SUP // 8, 128), F32),
        pltpu.VMEM((SUP // 8, 128), F32),
        pltpu.VMEM((SUP, 16), F32),
        pltpu.VMEM_SHARED((NPAD, 16), F32),
        pltpu.VMEM_SHARED((NPAD, 16), F32),
        pltpu.SemaphoreType.DMA,
        pltpu.SemaphoreType.DMA,
    ]

    @functools.partial(pl.kernel, out_type=out_type, mesh=_mesh(),
                       scratch_types=scratch,
                       compiler_params=pltpu.CompilerParams(
                           use_tc_tiling_on_sc=False, needs_layout_passes=False))
    def k(a_hbm, ew_hbm, src3_hbm, dst3_hbm, accp_hbm, cntp_hbm,
          sidx, didx, abuf0, abuf1, ewbuf0, ewbuf1, obuf, acc_sh, cnt_sh,
          sem_a, sem_m):
        _edge_body_common(a_hbm, None, ew_hbm, src3_hbm, dst3_hbm, accp_hbm,
                          cntp_hbm, sidx, didx, (abuf0, abuf1), None,
                          (ewbuf0, ewbuf1), obuf, acc_sh, cnt_sh,
                          sem_a, None, sem_m, with_b=False, with_cnt=True)

    return k


def _make_edge():
    out_type = jax.ShapeDtypeStruct((NC, NPAD, 16), F32)
    scratch = [
        pltpu.VMEM((NSUP, CHB, 128), jnp.int32),
        pltpu.VMEM((NSUP, CHB, 128), jnp.int32),
        pltpu.VMEM((SUP, 16), F32),
        pltpu.VMEM((SUP, 16), F32),
        pltpu.VMEM((SUP, 16), F32),
        pltpu.VMEM((SUP, 16), F32),
        pltpu.VMEM((SUP // 8, 128), F32),
        pltpu.VMEM((SUP // 8, 128), F32),
        pltpu.VMEM_SHARED((NPAD, 16), F32),
        pltpu.SemaphoreType.DMA,
        pltpu.SemaphoreType.DMA,
        pltpu.SemaphoreType.DMA,
    ]

    @functools.partial(pl.kernel, out_type=out_type, mesh=_mesh(),
                       scratch_types=scratch,
                       compiler_params=pltpu.CompilerParams(
                           use_tc_tiling_on_sc=False, needs_layout_passes=False))
    def k(a_hbm, b_hbm, ew_hbm, src3_hbm, dst3_hbm, accp_hbm,
          sidx, didx, abuf0, abuf1, bbuf0, bbuf1, ewbuf0, ewbuf1, acc_sh,
          sem_a, sem_b, sem_m):
        _edge_body_common(a_hbm, b_hbm, ew_hbm, src3_hbm, dst3_hbm, accp_hbm,
                          None, sidx, didx, (abuf0, abuf1), (bbuf0, bbuf1),
                          (ewbuf0, ewbuf1), None, acc_sh, None,
                          sem_a, sem_b, sem_m, with_b=True, with_cnt=False)

    return k


def _make_balance():
    out_type = (
        jax.ShapeDtypeStruct((E_PAD,), F32),
        jax.ShapeDtypeStruct((NW, NPAD), F32),
    )
    scratch = [
        pltpu.VMEM((NPAD,), F32),
        pltpu.VMEM((NPAD,), F32),
        pltpu.VMEM((EPW,), jnp.int32),
        pltpu.VMEM((EPW,), jnp.int32),
        pltpu.VMEM((EPW,), F32),
        pltpu.VMEM((EPW,), F32),
    ]

    @functools.partial(pl.kernel, out_type=out_type, mesh=_mesh(),
                       scratch_types=scratch,
                       compiler_params=pltpu.CompilerParams(
                           use_tc_tiling_on_sc=False, needs_layout_passes=False))
    def k(p_hbm, src_hbm, dst_hbm, ea_hbm, flow_hbm, netp_hbm,
          ptab, netacc, sbuf, dbuf, eabuf, fbuf):
        c = lax.axis_index("c")
        s = lax.axis_index("s")
        wid = c * NS + s
        base0 = wid * EPW
        pltpu.sync_copy(p_hbm, ptab)
        pltpu.sync_copy(src_hbm.at[pl.ds(base0, EPW)], sbuf)
        pltpu.sync_copy(dst_hbm.at[pl.ds(base0, EPW)], dbuf)
        pltpu.sync_copy(ea_hbm.at[pl.ds(base0, EPW)], eabuf)

        zero16 = jnp.zeros((16,), F32)

        @plsc.parallel_loop(0, NPAD // 16, 1, unroll=8)
        def zr(r):
            netacc[pl.ds(r * 16, 16)] = zero16

        lane = lax.iota(jnp.int32, 16)

        def v16(kk, carry):
            sv = sbuf[pl.ds(kk * 16, 16)]
            dv = dbuf[pl.ds(kk * 16, 16)]
            ps = plsc.load_gather(ptab, [sv])
            pd = plsc.load_gather(ptab, [dv])
            fl = (ps - pd) * eabuf[pl.ds(kk * 16, 16)]
            fbuf[pl.ds(kk * 16, 16)] = fl
            nfl = -fl
            for j in range(16):
                m = lane == j
                plsc.addupdate_scatter(netacc, [sv], nfl, mask=m)
                plsc.addupdate_scatter(netacc, [dv], fl, mask=m)
            return carry

        lax.fori_loop(0, EPW // 16, v16, 0, unroll=2)
        pltpu.sync_copy(fbuf, flow_hbm.at[pl.ds(base0, EPW)])
        pltpu.sync_copy(netacc, netp_hbm.at[wid])

    return k


_NBLK = NPAD // 2
_NGRID = NPAD // _NBLK
_EBLK = 4096
_EGRID = E_PAD // _EBLK


def _prep_kernel(x_ref, wx_ref, b1_ref, xw_ref, a0_ref):
    xw = jnp.dot(x_ref[...], wx_ref[...], preferred_element_type=F32)
    xw_ref[...] = xw
    a0_ref[...] = xw[:, 0:16] + b1_ref[0:1, :]


def _prep_call(x_pad, wx, b1_0):
    return pl.pallas_call(
        _prep_kernel,
        grid=(_NGRID,),
        in_specs=[
            pl.BlockSpec((_NBLK, DN), lambda i: (i, 0)),
            pl.BlockSpec((DN, DN), lambda i: (0, 0)),
            pl.BlockSpec((8, 16), lambda i: (0, 0)),
        ],
        out_specs=[
            pl.BlockSpec((_NBLK, DN), lambda i: (i, 0)),
            pl.BlockSpec((_NBLK, 16), lambda i: (i, 0)),
        ],
        out_shape=[
            jax.ShapeDtypeStruct((NPAD, DN), F32),
            jax.ShapeDtypeStruct((NPAD, 16), F32),
        ],
    )(x_pad, wx, b1_0)


def _ew_kernel(ea_ref, we_ref, *out_refs):
    ea = ea_ref[...]
    for i, o in enumerate(out_refs):
        o[...] = jnp.dot(ea, we_ref[i], preferred_element_type=F32)


def _ew_call(ea8, we_bd):
    eblk8 = _EBLK // 8
    return pl.pallas_call(
        _ew_kernel,
        grid=(_EGRID,),
        in_specs=[
            pl.BlockSpec((eblk8, 32), lambda i: (i, 0)),
            pl.BlockSpec((NCONV, 32, 128), lambda i: (0, 0, 0)),
        ],
        out_specs=[pl.BlockSpec((eblk8, 128), lambda i: (i, 0))
                   for _ in range(NCONV)],
        out_shape=[jax.ShapeDtypeStruct((E_PAD // 8, 128), F32)
                   for _ in range(NCONV)],
    )(ea8, we_bd)


def _node_common(accp, w2, b2):
    s = accp[0] + accp[1]
    return jnp.dot(s, w2[...], preferred_element_type=F32), b2[0:1, 0:LAT]


def _node0_kernel(accp_ref, cntp_ref, xw_ref, w2_ref, b2_ref, w1s_ref,
                  w1d_ref, b1_ref, a_ref, b_ref, cnt0_ref, cntc_ref):
    sw2, b2 = _node_common(accp_ref[...], w2_ref, b2_ref)
    cnt0 = (cntp_ref[0, :, 0] + cntp_ref[1, :, 0]).reshape(-1, 1)
    cntc = jnp.maximum(cnt0, 1.0)
    agg = (sw2 + cnt0 * b2) / cntc
    xx = jnp.maximum(agg, 0.0)
    a_ref[...] = (jnp.dot(xx, w1s_ref[0:LAT, :], preferred_element_type=F32)
                  + xw_ref[:, 16:32] + b1_ref[0:1, :])
    b_ref[...] = jnp.dot(xx, w1d_ref[0:LAT, :], preferred_element_type=F32)
    cnt0_ref[...] = cnt0
    cntc_ref[...] = cntc


def _node_mid_kernel(i, accp_ref, cnt0_ref, cntc_ref, xw_ref, w2_ref, b2_ref,
                     w1s_ref, w1d_ref, b1_ref, a_ref, b_ref):
    sw2, b2 = _node_common(accp_ref[...], w2_ref, b2_ref)
    agg = (sw2 + cnt0_ref[...] * b2) / cntc_ref[...]
    xx = jnp.maximum(agg, 0.0)
    a_ref[...] = (jnp.dot(xx, w1s_ref[0:LAT, :], preferred_element_type=F32)
                  + xw_ref[:, 16 * (i + 1):16 * (i + 2)] + b1_ref[0:1, :])
    b_ref[...] = jnp.dot(xx, w1d_ref[0:LAT, :], preferred_element_type=F32)


def _node_last_kernel(accp_ref, cnt0_ref, cntc_ref, w2_ref, b2_ref, wf_ref,
                      bf_ref, xlast_ref, p_ref, pm_ref):
    sw2, b2 = _node_common(accp_ref[...], w2_ref, b2_ref)
    agg = (sw2 + cnt0_ref[...] * b2) / cntc_ref[...]
    xx = jnp.maximum(agg, 0.0)
    p = jnp.maximum(
        jnp.dot(xx, wf_ref[0:LAT, :], preferred_element_type=F32)
        + bf_ref[0:1, :], 0.0)
    p_ref[...] = p
    xl = xlast_ref[...]
    pm_ref[...] = jnp.where(xl != 0.0, xl, p)


def _node0_call(accp, cntp, xw, w2, b2, w1s, w1d, b1):
    full = lambda shape: pl.BlockSpec(shape, lambda i: tuple(0 for _ in shape))
    return pl.pallas_call(
        _node0_kernel,
        grid=(_NGRID,),
        in_specs=[
            pl.BlockSpec((NC, _NBLK, 16), lambda i: (0, i, 0)),
            pl.BlockSpec((NC, _NBLK, 16), lambda i: (0, i, 0)),
            pl.BlockSpec((_NBLK, DN), lambda i: (i, 0)),
            full((16, LAT)), full((8, 8)), full((8, 16)), full((8, 16)),
            full((8, 16)),
        ],
        out_specs=[
            pl.BlockSpec((_NBLK, 16), lambda i: (i, 0)),
            pl.BlockSpec((_NBLK, 16), lambda i: (i, 0)),
            pl.BlockSpec((_NBLK, 1), lambda i: (i, 0)),
            pl.BlockSpec((_NBLK, 1), lambda i: (i, 0)),
        ],
        out_shape=[
            jax.ShapeDtypeStruct((NPAD, 16), F32),
            jax.ShapeDtypeStruct((NPAD, 16), F32),
            jax.ShapeDtypeStruct((NPAD, 1), F32),
            jax.ShapeDtypeStruct((NPAD, 1), F32),
        ],
    )(accp, cntp, xw, w2, b2, w1s, w1d, b1)


def _node_mid_call(i, accp, cnt0, cntc, xw, w2, b2, w1s, w1d, b1):
    full = lambda shape: pl.BlockSpec(shape, lambda i_: tuple(0 for _ in shape))
    return pl.pallas_call(
        functools.partial(_node_mid_kernel, i),
        grid=(_NGRID,),
        in_specs=[
            pl.BlockSpec((NC, _NBLK, 16), lambda i_: (0, i_, 0)),
            pl.BlockSpec((_NBLK, 1), lambda i_: (i_, 0)),
            pl.BlockSpec((_NBLK, 1), lambda i_: (i_, 0)),
            pl.BlockSpec((_NBLK, DN), lambda i_: (i_, 0)),
            full((16, LAT)), full((8, 8)), full((8, 16)), full((8, 16)),
            full((8, 16)),
        ],
        out_specs=[
            pl.BlockSpec((_NBLK, 16), lambda i_: (i_, 0)),
            pl.BlockSpec((_NBLK, 16), lambda i_: (i_, 0)),
        ],
        out_shape=[
            jax.ShapeDtypeStruct((NPAD, 16), F32),
            jax.ShapeDtypeStruct((NPAD, 16), F32),
        ],
    )(accp, cnt0, cntc, xw, w2, b2, w1s, w1d, b1)


def _node_last_call(accp, cnt0, cntc, w2, b2, wf, bf, xlast):
    full = lambda shape: pl.BlockSpec(shape, lambda i: tuple(0 for _ in shape))
    return pl.pallas_call(
        _node_last_kernel,
        grid=(_NGRID,),
        in_specs=[
            pl.BlockSpec((NC, _NBLK, 16), lambda i: (0, i, 0)),
            pl.BlockSpec((_NBLK, 1), lambda i: (i, 0)),
            pl.BlockSpec((_NBLK, 1), lambda i: (i, 0)),
            full((16, LAT)), full((8, 8)), full((8, 1)), full((8, 1)),
            pl.BlockSpec((_NBLK, 1), lambda i: (i, 0)),
        ],
        out_specs=[
            pl.BlockSpec((_NBLK, 1), lambda i: (i, 0)),
            pl.BlockSpec((_NBLK, 1), lambda i: (i, 0)),
        ],
        out_shape=[
            jax.ShapeDtypeStruct((NPAD, 1), F32),
            jax.ShapeDtypeStruct((NPAD, 1), F32),
        ],
    )(accp, cnt0, cntc, w2, b2, wf, bf, xlast)


def _imbal_kernel(netp_ref, p_ref, out_ref):
    net = p_ref[0:1, :] + jnp.sum(netp_ref[...], axis=0, keepdims=True)
    out_ref[...] = jnp.sum(jnp.abs(net)).reshape(1, 1)


def _imbal_call(netp, p_row):
    return pl.pallas_call(
        _imbal_kernel,
        grid=(1,),
        in_specs=[
            pl.BlockSpec((NW, NPAD), lambda i: (0, 0)),
            pl.BlockSpec((8, NPAD), lambda i: (0, 0)),
        ],
        out_specs=pl.BlockSpec((1, 1), lambda i: (0, 0)),
        out_shape=jax.ShapeDtypeStruct((1, 1), F32),
    )(netp, p_row)


def kernel(x, edge_index, edge_attr, W1, b1, W2, b2, Wf, bf):
    src = edge_index[0]
    dst = edge_index[1]
    npad_e = E_PAD - E
    src_p = jnp.concatenate([src, jnp.zeros((npad_e,), jnp.int32)])
    dst_p = jnp.concatenate([dst, jnp.full((npad_e,), N, jnp.int32)])
    src3_p = src_p.reshape(E_PAD // SUP, CHB, 128)
    dst3_p = dst_p.reshape(E_PAD // SUP, CHB, 128)
    ea_p = jnp.concatenate([edge_attr, jnp.zeros((npad_e, DE), F32)], axis=0)
    ea8_p = ea_p.reshape(E_PAD // 8, 8 * DE)
    x_pad = jnp.concatenate([x, jnp.zeros((NPAD - N, DN), F32)], axis=0)
    xlast_pad = x_pad[:, DN - 1:DN]

    w1s = W1[:, 0:LAT, :]
    w1d = W1[:, LAT:2 * LAT, :]
    w1x = W1[:, 2 * LAT:2 * LAT + DN, :]
    w1e = W1[:, 2 * LAT + DN:, :]
    wx_all = jnp.transpose(w1x, (1, 0, 2)).reshape(DN, NCONV * 16)
    we_bd = jax.vmap(
        lambda w: jnp.kron(jnp.eye(8, dtype=F32), w))(w1e)
    w1s_p = jnp.concatenate([w1s, jnp.zeros((NCONV, 2, 16), F32)], axis=1)
    w1d_p = jnp.concatenate([w1d, jnp.zeros((NCONV, 2, 16), F32)], axis=1)
    b1_bc = jnp.broadcast_to(b1[:, None, :], (NCONV, 8, 16))
    b2_bc = jnp.broadcast_to(
        jnp.pad(b2, ((0, 0), (0, 2)))[:, None, :], (NCONV, 8, 8))
    wf_p = jnp.concatenate([Wf, jnp.zeros((2, 1), F32)], axis=0)
    bf_bc = jnp.broadcast_to(bf.reshape(1, 1), (8, 1))

    xw, a0 = _prep_call(x_pad, wx_all, b1_bc[0])
    ews = _ew_call(ea8_p, we_bd)

    edge0 = _make_edge0()
    edge = _make_edge()

    accp, cntp = edge0(a0, ews[0], src3_p, dst3_p)
    a_t, b_t, cnt0, cntc = _node0_call(
        accp, cntp, xw, W2[0], b2_bc[0], w1s_p[1], w1d_p[1], b1_bc[1])

    for i in range(1, NCONV - 1):
        accp = edge(a_t, b_t, ews[i], src3_p, dst3_p)
        a_t, b_t = _node_mid_call(
            i, accp, cnt0, cntc, xw, W2[i], b2_bc[i],
            w1s_p[i + 1], w1d_p[i + 1], b1_bc[i + 1])

    accp = edge(a_t, b_t, ews[NCONV - 1], src3_p, dst3_p)
    p_full, pm_full = _node_last_call(
        accp, cnt0, cntc, W2[NCONV - 1], b2_bc[NCONV - 1], wf_p, bf_bc,
        xlast_pad)

    p_tab = jnp.concatenate([pm_full[:N, 0], jnp.zeros((NPAD - N,), F32)])
    ea0_p = ea_p[:, 0]

    balance = _make_balance()
    flow_pad, netp = balance(p_tab, src_p, dst_p, ea0_p)

    p_row = jnp.broadcast_to(p_tab[None, :], (8, NPAD))
    imb = _imbal_call(netp, p_row)

    P = p_full[:N]
    flow = flow_pad[:E].reshape(E, 1)
    return (P, flow, imb.reshape(1))


# speedup vs baseline: 15.9941x; 1.0085x over previous
import functools

import jax
import jax.numpy as jnp
from jax import lax
from jax.experimental import pallas as pl
from jax.experimental.pallas import tpu as pltpu
from jax.experimental.pallas import tpu_sc as plsc

N = 10000
E = 320000
DN = 128
DE = 4
LAT = 6
H = 16
NCONV = 8

NC = 2
NS = 16
NW = NC * NS
NPAD = N + 112
SUP = 512
CHB = SUP // 128
EPW = 10240
E_PAD = EPW * NW
NSUP = EPW // SUP
NS0 = 16
NS1 = 24
NSMAX = max(NS0, NS1)
RPT = NPAD // NS

F32 = jnp.float32


def _mesh():
    return plsc.VectorSubcoreMesh(
        core_axis_name="c", subcore_axis_name="s", num_cores=NC, num_subcores=NS
    )


def _edge_body_common(a_hbm, b_hbm, ew_hbm, src3_hbm, dst3_hbm, accp_hbm,
                      cntp_hbm, sidx, didx, abufs, bbufs, ewbufs, obuf,
                      acc_sh, cnt_sh, sem_a, sem_b, sem_m, with_b, with_cnt):
    c = lax.axis_index("c")
    s = lax.axis_index("s")

    my_nsup = jnp.where(c == 0, NS0, NS1)
    row0 = jnp.where(c == 0, s * NS0, NS * NS0 + s * NS1)

    pltpu.sync_copy(src3_hbm.at[pl.ds(row0, NSMAX)], sidx)
    pltpu.sync_copy(dst3_hbm.at[pl.ds(row0, NSMAX)], didx)

    zero16 = jnp.zeros((16,), F32)
    az = abufs[0]

    @plsc.parallel_loop(0, SUP, 1, unroll=8)
    def zrow(r):
        az[r] = zero16

    pltpu.sync_copy(az, acc_sh.at[pl.ds(s * RPT, SUP)])
    pltpu.sync_copy(az.at[pl.ds(0, RPT - SUP)],
                    acc_sh.at[pl.ds(s * RPT + SUP, RPT - SUP)])
    if with_cnt:
        lane = lax.iota(jnp.int32, 16)
        onerow = jnp.where(lane == 0, 1.0, 0.0).astype(F32)

        @plsc.parallel_loop(0, SUP, 1, unroll=8)
        def orow(r):
            obuf[r] = onerow

        pltpu.sync_copy(az, cnt_sh.at[pl.ds(s * RPT, SUP)])
        pltpu.sync_copy(az.at[pl.ds(0, RPT - SUP)],
                        cnt_sh.at[pl.ds(s * RPT + SUP, RPT - SUP)])
    plsc.subcore_barrier()

    base0 = row0 * SUP

    def fill(g, bi):
        base = base0 + g * SUP
        pltpu.async_copy(ew_hbm.at[pl.ds(base // 8, SUP // 8)], ewbufs[bi],
                         sem_m)
        for j in range(CHB):
            pltpu.async_copy(a_hbm.at[sidx.at[g, j]],
                             abufs[bi].at[pl.ds(j * 128, 128)], sem_a)
        if with_b:
            for j in range(CHB):
                pltpu.async_copy(b_hbm.at[didx.at[g, j]],
                                 bbufs[bi].at[pl.ds(j * 128, 128)], sem_b)

    def process(g, bi):
        pltpu.make_async_copy(ew_hbm.at[pl.ds(0, SUP // 8)], ewbufs[bi],
                              sem_m).wait()
        pltpu.make_async_copy(a_hbm.at[pl.ds(0, SUP)], abufs[bi],
                              sem_a).wait()
        ab = abufs[bi]
        eb = ewbufs[bi]
        if with_b:
            pltpu.make_async_copy(a_hbm.at[pl.ds(0, SUP)], bbufs[bi],
                                  sem_b).wait()
            bb = bbufs[bi]

            @plsc.parallel_loop(0, SUP // 8, 1, unroll=2)
            def comp(e8):
                for j in range(8):
                    e = e8 * 8 + j
                    ab[e] = jnp.maximum(
                        ab[e] + bb[e] + eb[e8, pl.ds(j * 16, 16)], 0.0)
        else:
            @plsc.parallel_loop(0, SUP // 8, 1, unroll=2)
            def comp(e8):
                for j in range(8):
                    e = e8 * 8 + j
                    ab[e] = jnp.maximum(
                        ab[e] + eb[e8, pl.ds(j * 16, 16)], 0.0)

        for j in range(CHB):
            pltpu.sync_copy(ab.at[pl.ds(j * 128, 128)],
                            acc_sh.at[didx.at[g, j]], add=True)
        if with_cnt:
            for j in range(CHB):
                pltpu.sync_copy(obuf.at[pl.ds(j * 128, 128)],
                                cnt_sh.at[didx.at[g, j]], add=True)

    fill(0, 0)
    fill(1, 1)

    def lbody(i, carry):
        g0 = 2 * i
        process(g0, 0)
        fill(g0 + 2, 0)
        process(g0 + 1, 1)
        fill(g0 + 3, 1)
        return carry

    lax.fori_loop(0, my_nsup // 2 - 1, lbody, 0)
    process(my_nsup - 2, 0)
    process(my_nsup - 1, 1)

    plsc.subcore_barrier()
    pltpu.sync_copy(acc_sh.at[pl.ds(s * RPT, RPT)],
                    accp_hbm.at[c, pl.ds(s * RPT, RPT)])
    if with_cnt:
        pltpu.sync_copy(cnt_sh.at[pl.ds(s * RPT, RPT)],
                        cntp_hbm.at[c, pl.ds(s * RPT, RPT)])


def _make_edge0():
    out_type = (
        jax.ShapeDtypeStruct((NC, NPAD, 16), F32),
        jax.ShapeDtypeStruct((NC, NPAD, 16), F32),
    )
    scratch = [
        pltpu.VMEM((NSMAX, CHB, 128), jnp.int32),
        pltpu.VMEM((NSMAX, CHB, 128), jnp.int32),
        pltpu.VMEM((SUP, 16), F32),
        pltpu.VMEM((SUP, 16), F32),
        pltpu.VMEM((SUP // 8, 128), F32),
        pltpu.VMEM((SUP // 8, 128), F32),
        pltpu.VMEM((SUP, 16), F32),
        pltpu.VMEM_SHARED((NPAD, 16), F32),
        pltpu.VMEM_SHARED((NPAD, 16), F32),
        pltpu.SemaphoreType.DMA,
        pltpu.SemaphoreType.DMA,
    ]

    @functools.partial(pl.kernel, out_type=out_type, mesh=_mesh(),
                       scratch_types=scratch,
                       compiler_params=pltpu.CompilerParams(
                           use_tc_tiling_on_sc=False, needs_layout_passes=False))
    def k(a_hbm, ew_hbm, src3_hbm, dst3_hbm, accp_hbm, cntp_hbm,
          sidx, didx, abuf0, abuf1, ewbuf0, ewbuf1, obuf, acc_sh, cnt_sh,
          sem_a, sem_m):
        _edge_body_common(a_hbm, None, ew_hbm, src3_hbm, dst3_hbm, accp_hbm,
                          cntp_hbm, sidx, didx, (abuf0, abuf1), None,
                          (ewbuf0, ewbuf1), obuf, acc_sh, cnt_sh,
                          sem_a, None, sem_m, with_b=False, with_cnt=True)

    return k


def _make_edge():
    out_type = jax.ShapeDtypeStruct((NC, NPAD, 16), F32)
    scratch = [
        pltpu.VMEM((NSMAX, CHB, 128), jnp.int32),
        pltpu.VMEM((NSMAX, CHB, 128), jnp.int32),
        pltpu.VMEM((SUP, 16), F32),
        pltpu.VMEM((SUP, 16), F32),
        pltpu.VMEM((SUP, 16), F32),
        pltpu.VMEM((SUP, 16), F32),
        pltpu.VMEM((SUP // 8, 128), F32),
        pltpu.VMEM((SUP // 8, 128), F32),
        pltpu.VMEM_SHARED((NPAD, 16), F32),
        pltpu.SemaphoreType.DMA,
        pltpu.SemaphoreType.DMA,
        pltpu.SemaphoreType.DMA,
    ]

    @functools.partial(pl.kernel, out_type=out_type, mesh=_mesh(),
                       scratch_types=scratch,
                       compiler_params=pltpu.CompilerParams(
                           use_tc_tiling_on_sc=False, needs_layout_passes=False))
    def k(a_hbm, b_hbm, ew_hbm, src3_hbm, dst3_hbm, accp_hbm,
          sidx, didx, abuf0, abuf1, bbuf0, bbuf1, ewbuf0, ewbuf1, acc_sh,
          sem_a, sem_b, sem_m):
        _edge_body_common(a_hbm, b_hbm, ew_hbm, src3_hbm, dst3_hbm, accp_hbm,
                          None, sidx, didx, (abuf0, abuf1), (bbuf0, bbuf1),
                          (ewbuf0, ewbuf1), None, acc_sh, None,
                          sem_a, sem_b, sem_m, with_b=True, with_cnt=False)

    return k


def _make_balance():
    out_type = (
        jax.ShapeDtypeStruct((E_PAD,), F32),
        jax.ShapeDtypeStruct((NW, NPAD), F32),
    )
    scratch = [
        pltpu.VMEM((NPAD,), F32),
        pltpu.VMEM((NPAD,), F32),
        pltpu.VMEM((EPW,), jnp.int32),
        pltpu.VMEM((EPW,), jnp.int32),
        pltpu.VMEM((EPW,), F32),
        pltpu.VMEM((EPW,), F32),
    ]

    @functools.partial(pl.kernel, out_type=out_type, mesh=_mesh(),
                       scratch_types=scratch,
                       compiler_params=pltpu.CompilerParams(
                           use_tc_tiling_on_sc=False, needs_layout_passes=False))
    def k(p_hbm, src_hbm, dst_hbm, ea_hbm, flow_hbm, netp_hbm,
          ptab, netacc, sbuf, dbuf, eabuf, fbuf):
        c = lax.axis_index("c")
        s = lax.axis_index("s")
        wid = c * NS + s
        base0 = wid * EPW
        pltpu.sync_copy(p_hbm, ptab)
        pltpu.sync_copy(src_hbm.at[pl.ds(base0, EPW)], sbuf)
        pltpu.sync_copy(dst_hbm.at[pl.ds(base0, EPW)], dbuf)
        pltpu.sync_copy(ea_hbm.at[pl.ds(base0, EPW)], eabuf)

        zero16 = jnp.zeros((16,), F32)

        @plsc.parallel_loop(0, NPAD // 16, 1, unroll=8)
        def zr(r):
            netacc[pl.ds(r * 16, 16)] = zero16

        lane = lax.iota(jnp.int32, 16)

        def v16(kk, carry):
            sv = sbuf[pl.ds(kk * 16, 16)]
            dv = dbuf[pl.ds(kk * 16, 16)]
            ps = plsc.load_gather(ptab, [sv])
            pd = plsc.load_gather(ptab, [dv])
            fl = (ps - pd) * eabuf[pl.ds(kk * 16, 16)]
            fbuf[pl.ds(kk * 16, 16)] = fl
            nfl = -fl
            for j in range(16):
                m = lane == j
                plsc.addupdate_scatter(netacc, [sv], nfl, mask=m)
                plsc.addupdate_scatter(netacc, [dv], fl, mask=m)
            return carry

        lax.fori_loop(0, EPW // 16, v16, 0, unroll=2)
        pltpu.sync_copy(fbuf, flow_hbm.at[pl.ds(base0, EPW)])
        pltpu.sync_copy(netacc, netp_hbm.at[wid])

    return k


_NBLK = NPAD // 2
_NGRID = NPAD // _NBLK
_EBLK = 4096
_EGRID = E_PAD // _EBLK


def _prep_kernel(x_ref, wx_ref, b1_ref, xw_ref, a0_ref):
    xw = jnp.dot(x_ref[...], wx_ref[...], preferred_element_type=F32)
    xw_ref[...] = xw
    a0_ref[...] = xw[:, 0:16] + b1_ref[0:1, :]


def _prep_call(x_pad, wx, b1_0):
    return pl.pallas_call(
        _prep_kernel,
        grid=(_NGRID,),
        in_specs=[
            pl.BlockSpec((_NBLK, DN), lambda i: (i, 0)),
            pl.BlockSpec((DN, DN), lambda i: (0, 0)),
            pl.BlockSpec((8, 16), lambda i: (0, 0)),
        ],
        out_specs=[
            pl.BlockSpec((_NBLK, DN), lambda i: (i, 0)),
            pl.BlockSpec((_NBLK, 16), lambda i: (i, 0)),
        ],
        out_shape=[
            jax.ShapeDtypeStruct((NPAD, DN), F32),
            jax.ShapeDtypeStruct((NPAD, 16), F32),
        ],
    )(x_pad, wx, b1_0)


def _ew_kernel(ea_ref, we_ref, *out_refs):
    ea = ea_ref[...]
    for i, o in enumerate(out_refs):
        o[...] = jnp.dot(ea, we_ref[i], preferred_element_type=F32)


def _ew_call(ea8, we_bd):
    eblk8 = _EBLK // 8
    return pl.pallas_call(
        _ew_kernel,
        grid=(_EGRID,),
        in_specs=[
            pl.BlockSpec((eblk8, 32), lambda i: (i, 0)),
            pl.BlockSpec((NCONV, 32, 128), lambda i: (0, 0, 0)),
        ],
        out_specs=[pl.BlockSpec((eblk8, 128), lambda i: (i, 0))
                   for _ in range(NCONV)],
        out_shape=[jax.ShapeDtypeStruct((E_PAD // 8, 128), F32)
                   for _ in range(NCONV)],
    )(ea8, we_bd)


def _node_common(accp, w2, b2):
    s = accp[0] + accp[1]
    return jnp.dot(s, w2[...], preferred_element_type=F32), b2[0:1, 0:LAT]


def _node0_kernel(accp_ref, cntp_ref, xw_ref, w2_ref, b2_ref, w1s_ref,
                  w1d_ref, b1_ref, a_ref, b_ref, cnt0_ref, cntc_ref):
    sw2, b2 = _node_common(accp_ref[...], w2_ref, b2_ref)
    cnt0 = (cntp_ref[0, :, 0] + cntp_ref[1, :, 0]).reshape(-1, 1)
    cntc = jnp.maximum(cnt0, 1.0)
    agg = (sw2 + cnt0 * b2) / cntc
    xx = jnp.maximum(agg, 0.0)
    a_ref[...] = (jnp.dot(xx, w1s_ref[0:LAT, :], preferred_element_type=F32)
                  + xw_ref[:, 16:32] + b1_ref[0:1, :])
    b_ref[...] = jnp.dot(xx, w1d_ref[0:LAT, :], preferred_element_type=F32)
    cnt0_ref[...] = cnt0
    cntc_ref[...] = cntc


def _node_mid_kernel(i, accp_ref, cnt0_ref, cntc_ref, xw_ref, w2_ref, b2_ref,
                     w1s_ref, w1d_ref, b1_ref, a_ref, b_ref):
    sw2, b2 = _node_common(accp_ref[...], w2_ref, b2_ref)
    agg = (sw2 + cnt0_ref[...] * b2) / cntc_ref[...]
    xx = jnp.maximum(agg, 0.0)
    a_ref[...] = (jnp.dot(xx, w1s_ref[0:LAT, :], preferred_element_type=F32)
                  + xw_ref[:, 16 * (i + 1):16 * (i + 2)] + b1_ref[0:1, :])
    b_ref[...] = jnp.dot(xx, w1d_ref[0:LAT, :], preferred_element_type=F32)


def _node_last_kernel(accp_ref, cnt0_ref, cntc_ref, w2_ref, b2_ref, wf_ref,
                      bf_ref, xlast_ref, p_ref, pm_ref):
    sw2, b2 = _node_common(accp_ref[...], w2_ref, b2_ref)
    agg = (sw2 + cnt0_ref[...] * b2) / cntc_ref[...]
    xx = jnp.maximum(agg, 0.0)
    p = jnp.maximum(
        jnp.dot(xx, wf_ref[0:LAT, :], preferred_element_type=F32)
        + bf_ref[0:1, :], 0.0)
    p_ref[...] = p
    xl = xlast_ref[...]
    pm_ref[...] = jnp.where(xl != 0.0, xl, p)


def _node0_call(accp, cntp, xw, w2, b2, w1s, w1d, b1):
    full = lambda shape: pl.BlockSpec(shape, lambda i: tuple(0 for _ in shape))
    return pl.pallas_call(
        _node0_kernel,
        grid=(_NGRID,),
        in_specs=[
            pl.BlockSpec((NC, _NBLK, 16), lambda i: (0, i, 0)),
            pl.BlockSpec((NC, _NBLK, 16), lambda i: (0, i, 0)),
            pl.BlockSpec((_NBLK, DN), lambda i: (i, 0)),
            full((16, LAT)), full((8, 8)), full((8, 16)), full((8, 16)),
            full((8, 16)),
        ],
        out_specs=[
            pl.BlockSpec((_NBLK, 16), lambda i: (i, 0)),
            pl.BlockSpec((_NBLK, 16), lambda i: (i, 0)),
            pl.BlockSpec((_NBLK, 1), lambda i: (i, 0)),
            pl.BlockSpec((_NBLK, 1), lambda i: (i, 0)),
        ],
        out_shape=[
            jax.ShapeDtypeStruct((NPAD, 16), F32),
            jax.ShapeDtypeStruct((NPAD, 16), F32),
            jax.ShapeDtypeStruct((NPAD, 1), F32),
            jax.ShapeDtypeStruct((NPAD, 1), F32),
        ],
    )(accp, cntp, xw, w2, b2, w1s, w1d, b1)


def _node_mid_call(i, accp, cnt0, cntc, xw, w2, b2, w1s, w1d, b1):
    full = lambda shape: pl.BlockSpec(shape, lambda i_: tuple(0 for _ in shape))
    return pl.pallas_call(
        functools.partial(_node_mid_kernel, i),
        grid=(_NGRID,),
        in_specs=[
            pl.BlockSpec((NC, _NBLK, 16), lambda i_: (0, i_, 0)),
            pl.BlockSpec((_NBLK, 1), lambda i_: (i_, 0)),
            pl.BlockSpec((_NBLK, 1), lambda i_: (i_, 0)),
            pl.BlockSpec((_NBLK, DN), lambda i_: (i_, 0)),
            full((16, LAT)), full((8, 8)), full((8, 16)), full((8, 16)),
            full((8, 16)),
        ],
        out_specs=[
            pl.BlockSpec((_NBLK, 16), lambda i_: (i_, 0)),
            pl.BlockSpec((_NBLK, 16), lambda i_: (i_, 0)),
        ],
        out_shape=[
            jax.ShapeDtypeStruct((NPAD, 16), F32),
            jax.ShapeDtypeStruct((NPAD, 16), F32),
        ],
    )(accp, cnt0, cntc, xw, w2, b2, w1s, w1d, b1)


def _node_last_call(accp, cnt0, cntc, w2, b2, wf, bf, xlast):
    full = lambda shape: pl.BlockSpec(shape, lambda i: tuple(0 for _ in shape))
    return pl.pallas_call(
        _node_last_kernel,
        grid=(_NGRID,),
        in_specs=[
            pl.BlockSpec((NC, _NBLK, 16), lambda i: (0, i, 0)),
            pl.BlockSpec((_NBLK, 1), lambda i: (i, 0)),
            pl.BlockSpec((_NBLK, 1), lambda i: (i, 0)),
            full((16, LAT)), full((8, 8)), full((8, 1)), full((8, 1)),
            pl.BlockSpec((_NBLK, 1), lambda i: (i, 0)),
        ],
        out_specs=[
            pl.BlockSpec((_NBLK, 1), lambda i: (i, 0)),
            pl.BlockSpec((_NBLK, 1), lambda i: (i, 0)),
        ],
        out_shape=[
            jax.ShapeDtypeStruct((NPAD, 1), F32),
            jax.ShapeDtypeStruct((NPAD, 1), F32),
        ],
    )(accp, cnt0, cntc, w2, b2, wf, bf, xlast)


def _imbal_kernel(netp_ref, p_ref, out_ref):
    net = p_ref[0:1, :] + jnp.sum(netp_ref[...], axis=0, keepdims=True)
    out_ref[...] = jnp.sum(jnp.abs(net)).reshape(1, 1)


def _imbal_call(netp, p_row):
    return pl.pallas_call(
        _imbal_kernel,
        grid=(1,),
        in_specs=[
            pl.BlockSpec((NW, NPAD), lambda i: (0, 0)),
            pl.BlockSpec((8, NPAD), lambda i: (0, 0)),
        ],
        out_specs=pl.BlockSpec((1, 1), lambda i: (0, 0)),
        out_shape=jax.ShapeDtypeStruct((1, 1), F32),
    )(netp, p_row)


def kernel(x, edge_index, edge_attr, W1, b1, W2, b2, Wf, bf):
    src = edge_index[0]
    dst = edge_index[1]
    npad_e = E_PAD - E
    src_p = jnp.concatenate([src, jnp.zeros((npad_e,), jnp.int32)])
    dst_p = jnp.concatenate([dst, jnp.full((npad_e,), N, jnp.int32)])
    src3_p = jnp.concatenate(
        [src_p, jnp.zeros((NSMAX * SUP,), jnp.int32)]).reshape(-1, CHB, 128)
    dst3_p = jnp.concatenate(
        [dst_p, jnp.full((NSMAX * SUP,), N, jnp.int32)]).reshape(-1, CHB, 128)
    ea_p = jnp.concatenate([edge_attr, jnp.zeros((npad_e, DE), F32)], axis=0)
    ea8_p = ea_p.reshape(E_PAD // 8, 8 * DE)
    x_pad = jnp.concatenate([x, jnp.zeros((NPAD - N, DN), F32)], axis=0)
    xlast_pad = x_pad[:, DN - 1:DN]

    w1s = W1[:, 0:LAT, :]
    w1d = W1[:, LAT:2 * LAT, :]
    w1x = W1[:, 2 * LAT:2 * LAT + DN, :]
    w1e = W1[:, 2 * LAT + DN:, :]
    wx_all = jnp.transpose(w1x, (1, 0, 2)).reshape(DN, NCONV * 16)
    we_bd = jax.vmap(
        lambda w: jnp.kron(jnp.eye(8, dtype=F32), w))(w1e)
    w1s_p = jnp.concatenate([w1s, jnp.zeros((NCONV, 2, 16), F32)], axis=1)
    w1d_p = jnp.concatenate([w1d, jnp.zeros((NCONV, 2, 16), F32)], axis=1)
    b1_bc = jnp.broadcast_to(b1[:, None, :], (NCONV, 8, 16))
    b2_bc = jnp.broadcast_to(
        jnp.pad(b2, ((0, 0), (0, 2)))[:, None, :], (NCONV, 8, 8))
    wf_p = jnp.concatenate([Wf, jnp.zeros((2, 1), F32)], axis=0)
    bf_bc = jnp.broadcast_to(bf.reshape(1, 1), (8, 1))

    xw, a0 = _prep_call(x_pad, wx_all, b1_bc[0])
    ews = _ew_call(ea8_p, we_bd)

    edge0 = _make_edge0()
    edge = _make_edge()

    accp, cntp = edge0(a0, ews[0], src3_p, dst3_p)
    a_t, b_t, cnt0, cntc = _node0_call(
        accp, cntp, xw, W2[0], b2_bc[0], w1s_p[1], w1d_p[1], b1_bc[1])

    for i in range(1, NCONV - 1):
        accp = edge(a_t, b_t, ews[i], src3_p, dst3_p)
        a_t, b_t = _node_mid_call(
            i, accp, cnt0, cntc, xw, W2[i], b2_bc[i],
            w1s_p[i + 1], w1d_p[i + 1], b1_bc[i + 1])

    accp = edge(a_t, b_t, ews[NCONV - 1], src3_p, dst3_p)
    p_full, pm_full = _node_last_call(
        accp, cnt0, cntc, W2[NCONV - 1], b2_bc[NCONV - 1], wf_p, bf_bc,
        xlast_pad)

    p_tab = jnp.concatenate([pm_full[:N, 0], jnp.zeros((NPAD - N,), F32)])
    ea0_p = ea_p[:, 0]

    balance = _make_balance()
    flow_pad, netp = balance(p_tab, src_p, dst_p, ea0_p)

    p_row = jnp.broadcast_to(p_tab[None, :], (8, NPAD))
    imb = _imbal_call(netp, p_row)

    P = p_full[:N]
    flow = flow_pad[:E].reshape(E, 1)
    return (P, flow, imb.reshape(1))


# speedup vs baseline: 17.2656x; 1.0795x over previous
import functools

import jax
import jax.numpy as jnp
from jax import lax
from jax.experimental import pallas as pl
from jax.experimental.pallas import tpu as pltpu
from jax.experimental.pallas import tpu_sc as plsc

N = 10000
E = 320000
DN = 128
DE = 4
LAT = 6
H = 16
NCONV = 8

NC = 2
NS = 16
NW = NC * NS
NPAD = N + 112
SUP = 512
CHB = SUP // 128
EPW = 10240
E_PAD = EPW * NW
NSUP = EPW // SUP
NS0 = 24
NS1 = 16
NSMAX = max(NS0, NS1)
RPT = NPAD // NS

F32 = jnp.float32


def _mesh():
    return plsc.VectorSubcoreMesh(
        core_axis_name="c", subcore_axis_name="s", num_cores=NC, num_subcores=NS
    )


def _edge_body_common(a_hbm, b_hbm, ew_hbm, src3_hbm, dst3_hbm, accp_hbm,
                      cntp_hbm, sidx, didx, abufs, bbufs, ewbufs, obuf,
                      acc_sh, cnt_sh, sem_a, sem_b, sem_m, with_b, with_cnt):
    c = lax.axis_index("c")
    s = lax.axis_index("s")

    my_nsup = jnp.where(c == 0, NS0, NS1)
    row0 = jnp.where(c == 0, s * NS0, NS * NS0 + s * NS1)

    pltpu.sync_copy(src3_hbm.at[pl.ds(row0, NSMAX)], sidx)
    pltpu.sync_copy(dst3_hbm.at[pl.ds(row0, NSMAX)], didx)

    zero16 = jnp.zeros((16,), F32)
    az = abufs[0]

    @plsc.parallel_loop(0, SUP, 1, unroll=8)
    def zrow(r):
        az[r] = zero16

    pltpu.sync_copy(az, acc_sh.at[pl.ds(s * RPT, SUP)])
    pltpu.sync_copy(az.at[pl.ds(0, RPT - SUP)],
                    acc_sh.at[pl.ds(s * RPT + SUP, RPT - SUP)])
    if with_cnt:
        lane = lax.iota(jnp.int32, 16)
        onerow = jnp.where(lane == 0, 1.0, 0.0).astype(F32)

        @plsc.parallel_loop(0, SUP, 1, unroll=8)
        def orow(r):
            obuf[r] = onerow

        pltpu.sync_copy(az, cnt_sh.at[pl.ds(s * RPT, SUP)])
        pltpu.sync_copy(az.at[pl.ds(0, RPT - SUP)],
                        cnt_sh.at[pl.ds(s * RPT + SUP, RPT - SUP)])
    plsc.subcore_barrier()

    base0 = row0 * SUP

    def fill(g, bi):
        base = base0 + g * SUP
        pltpu.async_copy(ew_hbm.at[pl.ds(base // 8, SUP // 8)], ewbufs[bi],
                         sem_m)
        for j in range(CHB):
            pltpu.async_copy(a_hbm.at[sidx.at[g, j]],
                             abufs[bi].at[pl.ds(j * 128, 128)], sem_a)
        if with_b:
            for j in range(CHB):
                pltpu.async_copy(b_hbm.at[didx.at[g, j]],
                                 bbufs[bi].at[pl.ds(j * 128, 128)], sem_b)

    def process(g, bi):
        pltpu.make_async_copy(ew_hbm.at[pl.ds(0, SUP // 8)], ewbufs[bi],
                              sem_m).wait()
        pltpu.make_async_copy(a_hbm.at[pl.ds(0, SUP)], abufs[bi],
                              sem_a).wait()
        ab = abufs[bi]
        eb = ewbufs[bi]
        if with_b:
            pltpu.make_async_copy(a_hbm.at[pl.ds(0, SUP)], bbufs[bi],
                                  sem_b).wait()
            bb = bbufs[bi]

            @plsc.parallel_loop(0, SUP // 8, 1, unroll=2)
            def comp(e8):
                for j in range(8):
                    e = e8 * 8 + j
                    ab[e] = jnp.maximum(
                        ab[e] + bb[e] + eb[e8, pl.ds(j * 16, 16)], 0.0)
        else:
            @plsc.parallel_loop(0, SUP // 8, 1, unroll=2)
            def comp(e8):
                for j in range(8):
                    e = e8 * 8 + j
                    ab[e] = jnp.maximum(
                        ab[e] + eb[e8, pl.ds(j * 16, 16)], 0.0)

        for j in range(CHB):
            pltpu.sync_copy(ab.at[pl.ds(j * 128, 128)],
                            acc_sh.at[didx.at[g, j]], add=True)
        if with_cnt:
            for j in range(CHB):
                pltpu.sync_copy(obuf.at[pl.ds(j * 128, 128)],
                                cnt_sh.at[didx.at[g, j]], add=True)

    fill(0, 0)
    fill(1, 1)

    def lbody(i, carry):
        g0 = 2 * i
        process(g0, 0)
        fill(g0 + 2, 0)
        process(g0 + 1, 1)
        fill(g0 + 3, 1)
        return carry

    lax.fori_loop(0, my_nsup // 2 - 1, lbody, 0)
    process(my_nsup - 2, 0)
    process(my_nsup - 1, 1)

    plsc.subcore_barrier()
    pltpu.sync_copy(acc_sh.at[pl.ds(s * RPT, RPT)],
                    accp_hbm.at[c, pl.ds(s * RPT, RPT)])
    if with_cnt:
        pltpu.sync_copy(cnt_sh.at[pl.ds(s * RPT, RPT)],
                        cntp_hbm.at[c, pl.ds(s * RPT, RPT)])


def _make_edge0():
    out_type = (
        jax.ShapeDtypeStruct((NC, NPAD, 16), F32),
        jax.ShapeDtypeStruct((NC, NPAD, 16), F32),
    )
    scratch = [
        pltpu.VMEM((NSMAX, CHB, 128), jnp.int32),
        pltpu.VMEM((NSMAX, CHB, 128), jnp.int32),
        pltpu.VMEM((SUP, 16), F32),
        pltpu.VMEM((SUP, 16), F32),
        pltpu.VMEM((SUP // 8, 128), F32),
        pltpu.VMEM((SUP // 8, 128), F32),
        pltpu.VMEM((SUP, 16), F32),
        pltpu.VMEM_SHARED((NPAD, 16), F32),
        pltpu.VMEM_SHARED((NPAD, 16), F32),
        pltpu.SemaphoreType.DMA,
        pltpu.SemaphoreType.DMA,
    ]

    @functools.partial(pl.kernel, out_type=out_type, mesh=_mesh(),
                       scratch_types=scratch,
                       compiler_params=pltpu.CompilerParams(
                           use_tc_tiling_on_sc=False, needs_layout_passes=False))
    def k(a_hbm, ew_hbm, src3_hbm, dst3_hbm, accp_hbm, cntp_hbm,
          sidx, didx, abuf0, abuf1, ewbuf0, ewbuf1, obuf, acc_sh, cnt_sh,
          sem_a, sem_m):
        _edge_body_common(a_hbm, None, ew_hbm, src3_hbm, dst3_hbm, accp_hbm,
                          cntp_hbm, sidx, didx, (abuf0, abuf1), None,
                          (ewbuf0, ewbuf1), obuf, acc_sh, cnt_sh,
                          sem_a, None, sem_m, with_b=False, with_cnt=True)

    return k


def _make_edge():
    out_type = jax.ShapeDtypeStruct((NC, NPAD, 16), F32)
    scratch = [
        pltpu.VMEM((NSMAX, CHB, 128), jnp.int32),
        pltpu.VMEM((NSMAX, CHB, 128), jnp.int32),
        pltpu.VMEM((SUP, 16), F32),
        pltpu.VMEM((SUP, 16), F32),
        pltpu.VMEM((SUP, 16), F32),
        pltpu.VMEM((SUP, 16), F32),
        pltpu.VMEM((SUP // 8, 128), F32),
        pltpu.VMEM((SUP // 8, 128), F32),
        pltpu.VMEM_SHARED((NPAD, 16), F32),
        pltpu.SemaphoreType.DMA,
        pltpu.SemaphoreType.DMA,
        pltpu.SemaphoreType.DMA,
    ]

    @functools.partial(pl.kernel, out_type=out_type, mesh=_mesh(),
                       scratch_types=scratch,
                       compiler_params=pltpu.CompilerParams(
                           use_tc_tiling_on_sc=False, needs_layout_passes=False))
    def k(a_hbm, b_hbm, ew_hbm, src3_hbm, dst3_hbm, accp_hbm,
          sidx, didx, abuf0, abuf1, bbuf0, bbuf1, ewbuf0, ewbuf1, acc_sh,
          sem_a, sem_b, sem_m):
        _edge_body_common(a_hbm, b_hbm, ew_hbm, src3_hbm, dst3_hbm, accp_hbm,
                          None, sidx, didx, (abuf0, abuf1), (bbuf0, bbuf1),
                          (ewbuf0, ewbuf1), None, acc_sh, None,
                          sem_a, sem_b, sem_m, with_b=True, with_cnt=False)

    return k


def _make_balance():
    out_type = (
        jax.ShapeDtypeStruct((E_PAD,), F32),
        jax.ShapeDtypeStruct((NW, NPAD), F32),
    )
    scratch = [
        pltpu.VMEM((NPAD,), F32),
        pltpu.VMEM((NPAD,), F32),
        pltpu.VMEM((EPW,), jnp.int32),
        pltpu.VMEM((EPW,), jnp.int32),
        pltpu.VMEM((EPW,), F32),
        pltpu.VMEM((EPW,), F32),
    ]

    @functools.partial(pl.kernel, out_type=out_type, mesh=_mesh(),
                       scratch_types=scratch,
                       compiler_params=pltpu.CompilerParams(
                           use_tc_tiling_on_sc=False, needs_layout_passes=False))
    def k(p_hbm, src_hbm, dst_hbm, ea_hbm, flow_hbm, netp_hbm,
          ptab, netacc, sbuf, dbuf, eabuf, fbuf):
        c = lax.axis_index("c")
        s = lax.axis_index("s")
        wid = c * NS + s
        base0 = wid * EPW
        pltpu.sync_copy(p_hbm, ptab)
        pltpu.sync_copy(src_hbm.at[pl.ds(base0, EPW)], sbuf)
        pltpu.sync_copy(dst_hbm.at[pl.ds(base0, EPW)], dbuf)
        pltpu.sync_copy(ea_hbm.at[pl.ds(base0, EPW)], eabuf)

        zero16 = jnp.zeros((16,), F32)

        @plsc.parallel_loop(0, NPAD // 16, 1, unroll=8)
        def zr(r):
            netacc[pl.ds(r * 16, 16)] = zero16

        lane = lax.iota(jnp.int32, 16)

        def v16(kk, carry):
            sv = sbuf[pl.ds(kk * 16, 16)]
            dv = dbuf[pl.ds(kk * 16, 16)]
            ps = plsc.load_gather(ptab, [sv])
            pd = plsc.load_gather(ptab, [dv])
            fl = (ps - pd) * eabuf[pl.ds(kk * 16, 16)]
            fbuf[pl.ds(kk * 16, 16)] = fl
            nfl = -fl
            for j in range(16):
                m = lane == j
                plsc.addupdate_scatter(netacc, [sv], nfl, mask=m)
                plsc.addupdate_scatter(netacc, [dv], fl, mask=m)
            return carry

        lax.fori_loop(0, EPW // 16, v16, 0, unroll=2)
        pltpu.sync_copy(fbuf, flow_hbm.at[pl.ds(base0, EPW)])
        pltpu.sync_copy(netacc, netp_hbm.at[wid])

    return k


_NBLK = NPAD // 2
_NGRID = NPAD // _NBLK
_EBLK = 4096
_EGRID = E_PAD // _EBLK


def _prep_kernel(x_ref, wx_ref, b1_ref, xw_ref, a0_ref):
    xw = jnp.dot(x_ref[...], wx_ref[...], preferred_element_type=F32)
    xw_ref[...] = xw
    a0_ref[...] = xw[:, 0:16] + b1_ref[0:1, :]


def _prep_call(x_pad, wx, b1_0):
    return pl.pallas_call(
        _prep_kernel,
        grid=(_NGRID,),
        in_specs=[
            pl.BlockSpec((_NBLK, DN), lambda i: (i, 0)),
            pl.BlockSpec((DN, DN), lambda i: (0, 0)),
            pl.BlockSpec((8, 16), lambda i: (0, 0)),
        ],
        out_specs=[
            pl.BlockSpec((_NBLK, DN), lambda i: (i, 0)),
            pl.BlockSpec((_NBLK, 16), lambda i: (i, 0)),
        ],
        out_shape=[
            jax.ShapeDtypeStruct((NPAD, DN), F32),
            jax.ShapeDtypeStruct((NPAD, 16), F32),
        ],
    )(x_pad, wx, b1_0)


def _ew_kernel(ea_ref, we_ref, *out_refs):
    ea = ea_ref[...]
    for i, o in enumerate(out_refs):
        o[...] = jnp.dot(ea, we_ref[i], preferred_element_type=F32)


def _ew_call(ea8, we_bd):
    eblk8 = _EBLK // 8
    return pl.pallas_call(
        _ew_kernel,
        grid=(_EGRID,),
        in_specs=[
            pl.BlockSpec((eblk8, 32), lambda i: (i, 0)),
            pl.BlockSpec((NCONV, 32, 128), lambda i: (0, 0, 0)),
        ],
        out_specs=[pl.BlockSpec((eblk8, 128), lambda i: (i, 0))
                   for _ in range(NCONV)],
        out_shape=[jax.ShapeDtypeStruct((E_PAD // 8, 128), F32)
                   for _ in range(NCONV)],
    )(ea8, we_bd)


def _node_common(accp, w2, b2):
    s = accp[0] + accp[1]
    return jnp.dot(s, w2[...], preferred_element_type=F32), b2[0:1, 0:LAT]


def _node0_kernel(accp_ref, cntp_ref, xw_ref, w2_ref, b2_ref, w1s_ref,
                  w1d_ref, b1_ref, a_ref, b_ref, cnt0_ref, cntc_ref):
    sw2, b2 = _node_common(accp_ref[...], w2_ref, b2_ref)
    cnt0 = (cntp_ref[0, :, 0] + cntp_ref[1, :, 0]).reshape(-1, 1)
    cntc = jnp.maximum(cnt0, 1.0)
    agg = (sw2 + cnt0 * b2) / cntc
    xx = jnp.maximum(agg, 0.0)
    a_ref[...] = (jnp.dot(xx, w1s_ref[0:LAT, :], preferred_element_type=F32)
                  + xw_ref[:, 16:32] + b1_ref[0:1, :])
    b_ref[...] = jnp.dot(xx, w1d_ref[0:LAT, :], preferred_element_type=F32)
    cnt0_ref[...] = cnt0
    cntc_ref[...] = cntc


def _node_mid_kernel(i, accp_ref, cnt0_ref, cntc_ref, xw_ref, w2_ref, b2_ref,
                     w1s_ref, w1d_ref, b1_ref, a_ref, b_ref):
    sw2, b2 = _node_common(accp_ref[...], w2_ref, b2_ref)
    agg = (sw2 + cnt0_ref[...] * b2) / cntc_ref[...]
    xx = jnp.maximum(agg, 0.0)
    a_ref[...] = (jnp.dot(xx, w1s_ref[0:LAT, :], preferred_element_type=F32)
                  + xw_ref[:, 16 * (i + 1):16 * (i + 2)] + b1_ref[0:1, :])
    b_ref[...] = jnp.dot(xx, w1d_ref[0:LAT, :], preferred_element_type=F32)


def _node_last_kernel(accp_ref, cnt0_ref, cntc_ref, w2_ref, b2_ref, wf_ref,
                      bf_ref, xlast_ref, p_ref, pm_ref):
    sw2, b2 = _node_common(accp_ref[...], w2_ref, b2_ref)
    agg = (sw2 + cnt0_ref[...] * b2) / cntc_ref[...]
    xx = jnp.maximum(agg, 0.0)
    p = jnp.maximum(
        jnp.dot(xx, wf_ref[0:LAT, :], preferred_element_type=F32)
        + bf_ref[0:1, :], 0.0)
    p_ref[...] = p
    xl = xlast_ref[...]
    pm_ref[...] = jnp.where(xl != 0.0, xl, p)


def _node0_call(accp, cntp, xw, w2, b2, w1s, w1d, b1):
    full = lambda shape: pl.BlockSpec(shape, lambda i: tuple(0 for _ in shape))
    return pl.pallas_call(
        _node0_kernel,
        grid=(_NGRID,),
        in_specs=[
            pl.BlockSpec((NC, _NBLK, 16), lambda i: (0, i, 0)),
            pl.BlockSpec((NC, _NBLK, 16), lambda i: (0, i, 0)),
            pl.BlockSpec((_NBLK, DN), lambda i: (i, 0)),
            full((16, LAT)), full((8, 8)), full((8, 16)), full((8, 16)),
            full((8, 16)),
        ],
        out_specs=[
            pl.BlockSpec((_NBLK, 16), lambda i: (i, 0)),
            pl.BlockSpec((_NBLK, 16), lambda i: (i, 0)),
            pl.BlockSpec((_NBLK, 1), lambda i: (i, 0)),
            pl.BlockSpec((_NBLK, 1), lambda i: (i, 0)),
        ],
        out_shape=[
            jax.ShapeDtypeStruct((NPAD, 16), F32),
            jax.ShapeDtypeStruct((NPAD, 16), F32),
            jax.ShapeDtypeStruct((NPAD, 1), F32),
            jax.ShapeDtypeStruct((NPAD, 1), F32),
        ],
    )(accp, cntp, xw, w2, b2, w1s, w1d, b1)


def _node_mid_call(i, accp, cnt0, cntc, xw, w2, b2, w1s, w1d, b1):
    full = lambda shape: pl.BlockSpec(shape, lambda i_: tuple(0 for _ in shape))
    return pl.pallas_call(
        functools.partial(_node_mid_kernel, i),
        grid=(_NGRID,),
        in_specs=[
            pl.BlockSpec((NC, _NBLK, 16), lambda i_: (0, i_, 0)),
            pl.BlockSpec((_NBLK, 1), lambda i_: (i_, 0)),
            pl.BlockSpec((_NBLK, 1), lambda i_: (i_, 0)),
            pl.BlockSpec((_NBLK, DN), lambda i_: (i_, 0)),
            full((16, LAT)), full((8, 8)), full((8, 16)), full((8, 16)),
            full((8, 16)),
        ],
        out_specs=[
            pl.BlockSpec((_NBLK, 16), lambda i_: (i_, 0)),
            pl.BlockSpec((_NBLK, 16), lambda i_: (i_, 0)),
        ],
        out_shape=[
            jax.ShapeDtypeStruct((NPAD, 16), F32),
            jax.ShapeDtypeStruct((NPAD, 16), F32),
        ],
    )(accp, cnt0, cntc, xw, w2, b2, w1s, w1d, b1)


def _node_last_call(accp, cnt0, cntc, w2, b2, wf, bf, xlast):
    full = lambda shape: pl.BlockSpec(shape, lambda i: tuple(0 for _ in shape))
    return pl.pallas_call(
        _node_last_kernel,
        grid=(_NGRID,),
        in_specs=[
            pl.BlockSpec((NC, _NBLK, 16), lambda i: (0, i, 0)),
            pl.BlockSpec((_NBLK, 1), lambda i: (i, 0)),
            pl.BlockSpec((_NBLK, 1), lambda i: (i, 0)),
            full((16, LAT)), full((8, 8)), full((8, 1)), full((8, 1)),
            pl.BlockSpec((_NBLK, 1), lambda i: (i, 0)),
        ],
        out_specs=[
            pl.BlockSpec((_NBLK, 1), lambda i: (i, 0)),
            pl.BlockSpec((_NBLK, 1), lambda i: (i, 0)),
        ],
        out_shape=[
            jax.ShapeDtypeStruct((NPAD, 1), F32),
            jax.ShapeDtypeStruct((NPAD, 1), F32),
        ],
    )(accp, cnt0, cntc, w2, b2, wf, bf, xlast)


def _imbal_kernel(netp_ref, p_ref, out_ref):
    net = p_ref[0:1, :] + jnp.sum(netp_ref[...], axis=0, keepdims=True)
    out_ref[...] = jnp.sum(jnp.abs(net)).reshape(1, 1)


def _imbal_call(netp, p_row):
    return pl.pallas_call(
        _imbal_kernel,
        grid=(1,),
        in_specs=[
            pl.BlockSpec((NW, NPAD), lambda i: (0, 0)),
            pl.BlockSpec((8, NPAD), lambda i: (0, 0)),
        ],
        out_specs=pl.BlockSpec((1, 1), lambda i: (0, 0)),
        out_shape=jax.ShapeDtypeStruct((1, 1), F32),
    )(netp, p_row)


def kernel(x, edge_index, edge_attr, W1, b1, W2, b2, Wf, bf):
    src = edge_index[0]
    dst = edge_index[1]
    npad_e = E_PAD - E
    src_p = jnp.concatenate([src, jnp.zeros((npad_e,), jnp.int32)])
    dst_p = jnp.concatenate([dst, jnp.full((npad_e,), N, jnp.int32)])
    src3_p = jnp.concatenate(
        [src_p, jnp.zeros((NSMAX * SUP,), jnp.int32)]).reshape(-1, CHB, 128)
    dst3_p = jnp.concatenate(
        [dst_p, jnp.full((NSMAX * SUP,), N, jnp.int32)]).reshape(-1, CHB, 128)
    ea_p = jnp.concatenate([edge_attr, jnp.zeros((npad_e, DE), F32)], axis=0)
    ea8_p = ea_p.reshape(E_PAD // 8, 8 * DE)
    x_pad = jnp.concatenate([x, jnp.zeros((NPAD - N, DN), F32)], axis=0)
    xlast_pad = x_pad[:, DN - 1:DN]

    w1s = W1[:, 0:LAT, :]
    w1d = W1[:, LAT:2 * LAT, :]
    w1x = W1[:, 2 * LAT:2 * LAT + DN, :]
    w1e = W1[:, 2 * LAT + DN:, :]
    wx_all = jnp.transpose(w1x, (1, 0, 2)).reshape(DN, NCONV * 16)
    we_bd = jax.vmap(
        lambda w: jnp.kron(jnp.eye(8, dtype=F32), w))(w1e)
    w1s_p = jnp.concatenate([w1s, jnp.zeros((NCONV, 2, 16), F32)], axis=1)
    w1d_p = jnp.concatenate([w1d, jnp.zeros((NCONV, 2, 16), F32)], axis=1)
    b1_bc = jnp.broadcast_to(b1[:, None, :], (NCONV, 8, 16))
    b2_bc = jnp.broadcast_to(
        jnp.pad(b2, ((0, 0), (0, 2)))[:, None, :], (NCONV, 8, 8))
    wf_p = jnp.concatenate([Wf, jnp.zeros((2, 1), F32)], axis=0)
    bf_bc = jnp.broadcast_to(bf.reshape(1, 1), (8, 1))

    xw, a0 = _prep_call(x_pad, wx_all, b1_bc[0])
    ews = _ew_call(ea8_p, we_bd)

    edge0 = _make_edge0()
    edge = _make_edge()

    accp, cntp = edge0(a0, ews[0], src3_p, dst3_p)
    a_t, b_t, cnt0, cntc = _node0_call(
        accp, cntp, xw, W2[0], b2_bc[0], w1s_p[1], w1d_p[1], b1_bc[1])

    for i in range(1, NCONV - 1):
        accp = edge(a_t, b_t, ews[i], src3_p, dst3_p)
        a_t, b_t = _node_mid_call(
            i, accp, cnt0, cntc, xw, W2[i], b2_bc[i],
            w1s_p[i + 1], w1d_p[i + 1], b1_bc[i + 1])

    accp = edge(a_t, b_t, ews[NCONV - 1], src3_p, dst3_p)
    p_full, pm_full = _node_last_call(
        accp, cnt0, cntc, W2[NCONV - 1], b2_bc[NCONV - 1], wf_p, bf_bc,
        xlast_pad)

    p_tab = jnp.concatenate([pm_full[:N, 0], jnp.zeros((NPAD - N,), F32)])
    ea0_p = ea_p[:, 0]

    balance = _make_balance()
    flow_pad, netp = balance(p_tab, src_p, dst_p, ea0_p)

    p_row = jnp.broadcast_to(p_tab[None, :], (8, NPAD))
    imb = _imbal_call(netp, p_row)

    P = p_full[:N]
    flow = flow_pad[:E].reshape(E, 1)
    return (P, flow, imb.reshape(1))


# speedup vs baseline: 17.4624x; 1.0114x over previous
import functools

import jax
import jax.numpy as jnp
from jax import lax
from jax.experimental import pallas as pl
from jax.experimental.pallas import tpu as pltpu
from jax.experimental.pallas import tpu_sc as plsc

N = 10000
E = 320000
DN = 128
DE = 4
LAT = 6
H = 16
NCONV = 8

NC = 2
NS = 16
NW = NC * NS
NPAD = N + 112
SUP = 512
CHB = SUP // 128
EPW = 10240
E_PAD = EPW * NW
NSUP = EPW // SUP
NS0 = 26
NS1 = 14
NSMAX = max(NS0, NS1)
RPT = NPAD // NS

F32 = jnp.float32


def _mesh():
    return plsc.VectorSubcoreMesh(
        core_axis_name="c", subcore_axis_name="s", num_cores=NC, num_subcores=NS
    )


def _edge_body_common(a_hbm, b_hbm, ew_hbm, src3_hbm, dst3_hbm, accp_hbm,
                      cntp_hbm, sidx, didx, abufs, bbufs, ewbufs, obuf,
                      acc_sh, cnt_sh, sem_a, sem_b, sem_m, with_b, with_cnt):
    c = lax.axis_index("c")
    s = lax.axis_index("s")

    my_nsup = jnp.where(c == 0, NS0, NS1)
    row0 = jnp.where(c == 0, s * NS0, NS * NS0 + s * NS1)

    pltpu.sync_copy(src3_hbm.at[pl.ds(row0, NSMAX)], sidx)
    pltpu.sync_copy(dst3_hbm.at[pl.ds(row0, NSMAX)], didx)

    zero16 = jnp.zeros((16,), F32)
    az = abufs[0]

    @plsc.parallel_loop(0, SUP, 1, unroll=8)
    def zrow(r):
        az[r] = zero16

    pltpu.sync_copy(az, acc_sh.at[pl.ds(s * RPT, SUP)])
    pltpu.sync_copy(az.at[pl.ds(0, RPT - SUP)],
                    acc_sh.at[pl.ds(s * RPT + SUP, RPT - SUP)])
    if with_cnt:
        lane = lax.iota(jnp.int32, 16)
        onerow = jnp.where(lane == 0, 1.0, 0.0).astype(F32)

        @plsc.parallel_loop(0, SUP, 1, unroll=8)
        def orow(r):
            obuf[r] = onerow

        pltpu.sync_copy(az, cnt_sh.at[pl.ds(s * RPT, SUP)])
        pltpu.sync_copy(az.at[pl.ds(0, RPT - SUP)],
                        cnt_sh.at[pl.ds(s * RPT + SUP, RPT - SUP)])
    plsc.subcore_barrier()

    base0 = row0 * SUP

    def fill(g, bi):
        base = base0 + g * SUP
        pltpu.async_copy(ew_hbm.at[pl.ds(base // 8, SUP // 8)], ewbufs[bi],
                         sem_m)
        for j in range(CHB):
            pltpu.async_copy(a_hbm.at[sidx.at[g, j]],
                             abufs[bi].at[pl.ds(j * 128, 128)], sem_a)
        if with_b:
            for j in range(CHB):
                pltpu.async_copy(b_hbm.at[didx.at[g, j]],
                                 bbufs[bi].at[pl.ds(j * 128, 128)], sem_b)

    def process(g, bi):
        pltpu.make_async_copy(ew_hbm.at[pl.ds(0, SUP // 8)], ewbufs[bi],
                              sem_m).wait()
        pltpu.make_async_copy(a_hbm.at[pl.ds(0, SUP)], abufs[bi],
                              sem_a).wait()
        ab = abufs[bi]
        eb = ewbufs[bi]
        if with_b:
            pltpu.make_async_copy(a_hbm.at[pl.ds(0, SUP)], bbufs[bi],
                                  sem_b).wait()
            bb = bbufs[bi]

            @plsc.parallel_loop(0, SUP // 8, 1, unroll=2)
            def comp(e8):
                for j in range(8):
                    e = e8 * 8 + j
                    ab[e] = jnp.maximum(
                        ab[e] + bb[e] + eb[e8, pl.ds(j * 16, 16)], 0.0)
        else:
            @plsc.parallel_loop(0, SUP // 8, 1, unroll=2)
            def comp(e8):
                for j in range(8):
                    e = e8 * 8 + j
                    ab[e] = jnp.maximum(
                        ab[e] + eb[e8, pl.ds(j * 16, 16)], 0.0)

        for j in range(CHB):
            pltpu.sync_copy(ab.at[pl.ds(j * 128, 128)],
                            acc_sh.at[didx.at[g, j]], add=True)
        if with_cnt:
            for j in range(CHB):
                pltpu.sync_copy(obuf.at[pl.ds(j * 128, 128)],
                                cnt_sh.at[didx.at[g, j]], add=True)

    fill(0, 0)
    fill(1, 1)

    def lbody(i, carry):
        g0 = 2 * i
        process(g0, 0)
        fill(g0 + 2, 0)
        process(g0 + 1, 1)
        fill(g0 + 3, 1)
        return carry

    lax.fori_loop(0, my_nsup // 2 - 1, lbody, 0)
    process(my_nsup - 2, 0)
    process(my_nsup - 1, 1)

    plsc.subcore_barrier()
    pltpu.sync_copy(acc_sh.at[pl.ds(s * RPT, RPT)],
                    accp_hbm.at[c, pl.ds(s * RPT, RPT)])
    if with_cnt:
        pltpu.sync_copy(cnt_sh.at[pl.ds(s * RPT, RPT)],
                        cntp_hbm.at[c, pl.ds(s * RPT, RPT)])


def _make_edge0():
    out_type = (
        jax.ShapeDtypeStruct((NC, NPAD, 16), F32),
        jax.ShapeDtypeStruct((NC, NPAD, 16), F32),
    )
    scratch = [
        pltpu.VMEM((NSMAX, CHB, 128), jnp.int32),
        pltpu.VMEM((NSMAX, CHB, 128), jnp.int32),
        pltpu.VMEM((SUP, 16), F32),
        pltpu.VMEM((SUP, 16), F32),
        pltpu.VMEM((SUP // 8, 128), F32),
        pltpu.VMEM((SUP // 8, 128), F32),
        pltpu.VMEM((SUP, 16), F32),
        pltpu.VMEM_SHARED((NPAD, 16), F32),
        pltpu.VMEM_SHARED((NPAD, 16), F32),
        pltpu.SemaphoreType.DMA,
        pltpu.SemaphoreType.DMA,
    ]

    @functools.partial(pl.kernel, out_type=out_type, mesh=_mesh(),
                       scratch_types=scratch,
                       compiler_params=pltpu.CompilerParams(
                           use_tc_tiling_on_sc=False, needs_layout_passes=False))
    def k(a_hbm, ew_hbm, src3_hbm, dst3_hbm, accp_hbm, cntp_hbm,
          sidx, didx, abuf0, abuf1, ewbuf0, ewbuf1, obuf, acc_sh, cnt_sh,
          sem_a, sem_m):
        _edge_body_common(a_hbm, None, ew_hbm, src3_hbm, dst3_hbm, accp_hbm,
                          cntp_hbm, sidx, didx, (abuf0, abuf1), None,
                          (ewbuf0, ewbuf1), obuf, acc_sh, cnt_sh,
                          sem_a, None, sem_m, with_b=False, with_cnt=True)

    return k


def _make_edge():
    out_type = jax.ShapeDtypeStruct((NC, NPAD, 16), F32)
    scratch = [
        pltpu.VMEM((NSMAX, CHB, 128), jnp.int32),
        pltpu.VMEM((NSMAX, CHB, 128), jnp.int32),
        pltpu.VMEM((SUP, 16), F32),
        pltpu.VMEM((SUP, 16), F32),
        pltpu.VMEM((SUP, 16), F32),
        pltpu.VMEM((SUP, 16), F32),
        pltpu.VMEM((SUP // 8, 128), F32),
        pltpu.VMEM((SUP // 8, 128), F32),
        pltpu.VMEM_SHARED((NPAD, 16), F32),
        pltpu.SemaphoreType.DMA,
        pltpu.SemaphoreType.DMA,
        pltpu.SemaphoreType.DMA,
    ]

    @functools.partial(pl.kernel, out_type=out_type, mesh=_mesh(),
                       scratch_types=scratch,
                       compiler_params=pltpu.CompilerParams(
                           use_tc_tiling_on_sc=False, needs_layout_passes=False))
    def k(a_hbm, b_hbm, ew_hbm, src3_hbm, dst3_hbm, accp_hbm,
          sidx, didx, abuf0, abuf1, bbuf0, bbuf1, ewbuf0, ewbuf1, acc_sh,
          sem_a, sem_b, sem_m):
        _edge_body_common(a_hbm, b_hbm, ew_hbm, src3_hbm, dst3_hbm, accp_hbm,
                          None, sidx, didx, (abuf0, abuf1), (bbuf0, bbuf1),
                          (ewbuf0, ewbuf1), None, acc_sh, None,
                          sem_a, sem_b, sem_m, with_b=True, with_cnt=False)

    return k


def _make_balance():
    out_type = (
        jax.ShapeDtypeStruct((E_PAD,), F32),
        jax.ShapeDtypeStruct((NW, NPAD), F32),
    )
    scratch = [
        pltpu.VMEM((NPAD,), F32),
        pltpu.VMEM((NPAD,), F32),
        pltpu.VMEM((EPW,), jnp.int32),
        pltpu.VMEM((EPW,), jnp.int32),
        pltpu.VMEM((EPW,), F32),
        pltpu.VMEM((EPW,), F32),
    ]

    @functools.partial(pl.kernel, out_type=out_type, mesh=_mesh(),
                       scratch_types=scratch,
                       compiler_params=pltpu.CompilerParams(
                           use_tc_tiling_on_sc=False, needs_layout_passes=False))
    def k(p_hbm, src_hbm, dst_hbm, ea_hbm, flow_hbm, netp_hbm,
          ptab, netacc, sbuf, dbuf, eabuf, fbuf):
        c = lax.axis_index("c")
        s = lax.axis_index("s")
        wid = c * NS + s
        base0 = wid * EPW
        pltpu.sync_copy(p_hbm, ptab)
        pltpu.sync_copy(src_hbm.at[pl.ds(base0, EPW)], sbuf)
        pltpu.sync_copy(dst_hbm.at[pl.ds(base0, EPW)], dbuf)
        pltpu.sync_copy(ea_hbm.at[pl.ds(base0, EPW)], eabuf)

        zero16 = jnp.zeros((16,), F32)

        @plsc.parallel_loop(0, NPAD // 16, 1, unroll=8)
        def zr(r):
            netacc[pl.ds(r * 16, 16)] = zero16

        lane = lax.iota(jnp.int32, 16)

        def v16(kk, carry):
            sv = sbuf[pl.ds(kk * 16, 16)]
            dv = dbuf[pl.ds(kk * 16, 16)]
            ps = plsc.load_gather(ptab, [sv])
            pd = plsc.load_gather(ptab, [dv])
            fl = (ps - pd) * eabuf[pl.ds(kk * 16, 16)]
            fbuf[pl.ds(kk * 16, 16)] = fl
            nfl = -fl
            for j in range(16):
                m = lane == j
                plsc.addupdate_scatter(netacc, [sv], nfl, mask=m)
                plsc.addupdate_scatter(netacc, [dv], fl, mask=m)
            return carry

        lax.fori_loop(0, EPW // 16, v16, 0, unroll=2)
        pltpu.sync_copy(fbuf, flow_hbm.at[pl.ds(base0, EPW)])
        pltpu.sync_copy(netacc, netp_hbm.at[wid])

    return k


_NBLK = NPAD // 2
_NGRID = NPAD // _NBLK
_EBLK = 4096
_EGRID = E_PAD // _EBLK


def _prep_kernel(x_ref, wx_ref, b1_ref, xw_ref, a0_ref):
    xw = jnp.dot(x_ref[...], wx_ref[...], preferred_element_type=F32)
    xw_ref[...] = xw
    a0_ref[...] = xw[:, 0:16] + b1_ref[0:1, :]


def _prep_call(x_pad, wx, b1_0):
    return pl.pallas_call(
        _prep_kernel,
        grid=(_NGRID,),
        in_specs=[
            pl.BlockSpec((_NBLK, DN), lambda i: (i, 0)),
            pl.BlockSpec((DN, DN), lambda i: (0, 0)),
            pl.BlockSpec((8, 16), lambda i: (0, 0)),
        ],
        out_specs=[
            pl.BlockSpec((_NBLK, DN), lambda i: (i, 0)),
            pl.BlockSpec((_NBLK, 16), lambda i: (i, 0)),
        ],
        out_shape=[
            jax.ShapeDtypeStruct((NPAD, DN), F32),
            jax.ShapeDtypeStruct((NPAD, 16), F32),
        ],
    )(x_pad, wx, b1_0)


def _ew_kernel(ea_ref, we_ref, *out_refs):
    ea = ea_ref[...]
    for i, o in enumerate(out_refs):
        o[...] = jnp.dot(ea, we_ref[i], preferred_element_type=F32)


def _ew_call(ea8, we_bd):
    eblk8 = _EBLK // 8
    return pl.pallas_call(
        _ew_kernel,
        grid=(_EGRID,),
        in_specs=[
            pl.BlockSpec((eblk8, 32), lambda i: (i, 0)),
            pl.BlockSpec((NCONV, 32, 128), lambda i: (0, 0, 0)),
        ],
        out_specs=[pl.BlockSpec((eblk8, 128), lambda i: (i, 0))
                   for _ in range(NCONV)],
        out_shape=[jax.ShapeDtypeStruct((E_PAD // 8, 128), F32)
                   for _ in range(NCONV)],
    )(ea8, we_bd)


def _node_common(accp, w2, b2):
    s = accp[0] + accp[1]
    return jnp.dot(s, w2[...], preferred_element_type=F32), b2[0:1, 0:LAT]


def _node0_kernel(accp_ref, cntp_ref, xw_ref, w2_ref, b2_ref, w1s_ref,
                  w1d_ref, b1_ref, a_ref, b_ref, cnt0_ref, cntc_ref):
    sw2, b2 = _node_common(accp_ref[...], w2_ref, b2_ref)
    cnt0 = (cntp_ref[0, :, 0] + cntp_ref[1, :, 0]).reshape(-1, 1)
    cntc = jnp.maximum(cnt0, 1.0)
    agg = (sw2 + cnt0 * b2) / cntc
    xx = jnp.maximum(agg, 0.0)
    a_ref[...] = (jnp.dot(xx, w1s_ref[0:LAT, :], preferred_element_type=F32)
                  + xw_ref[:, 16:32] + b1_ref[0:1, :])
    b_ref[...] = jnp.dot(xx, w1d_ref[0:LAT, :], preferred_element_type=F32)
    cnt0_ref[...] = cnt0
    cntc_ref[...] = cntc


def _node_mid_kernel(i, accp_ref, cnt0_ref, cntc_ref, xw_ref, w2_ref, b2_ref,
                     w1s_ref, w1d_ref, b1_ref, a_ref, b_ref):
    sw2, b2 = _node_common(accp_ref[...], w2_ref, b2_ref)
    agg = (sw2 + cnt0_ref[...] * b2) / cntc_ref[...]
    xx = jnp.maximum(agg, 0.0)
    a_ref[...] = (jnp.dot(xx, w1s_ref[0:LAT, :], preferred_element_type=F32)
                  + xw_ref[:, 16 * (i + 1):16 * (i + 2)] + b1_ref[0:1, :])
    b_ref[...] = jnp.dot(xx, w1d_ref[0:LAT, :], preferred_element_type=F32)


def _node_last_kernel(accp_ref, cnt0_ref, cntc_ref, w2_ref, b2_ref, wf_ref,
                      bf_ref, xlast_ref, p_ref, pm_ref):
    sw2, b2 = _node_common(accp_ref[...], w2_ref, b2_ref)
    agg = (sw2 + cnt0_ref[...] * b2) / cntc_ref[...]
    xx = jnp.maximum(agg, 0.0)
    p = jnp.maximum(
        jnp.dot(xx, wf_ref[0:LAT, :], preferred_element_type=F32)
        + bf_ref[0:1, :], 0.0)
    p_ref[...] = p
    xl = xlast_ref[...]
    pm_ref[...] = jnp.where(xl != 0.0, xl, p)


def _node0_call(accp, cntp, xw, w2, b2, w1s, w1d, b1):
    full = lambda shape: pl.BlockSpec(shape, lambda i: tuple(0 for _ in shape))
    return pl.pallas_call(
        _node0_kernel,
        grid=(_NGRID,),
        in_specs=[
            pl.BlockSpec((NC, _NBLK, 16), lambda i: (0, i, 0)),
            pl.BlockSpec((NC, _NBLK, 16), lambda i: (0, i, 0)),
            pl.BlockSpec((_NBLK, DN), lambda i: (i, 0)),
            full((16, LAT)), full((8, 8)), full((8, 16)), full((8, 16)),
            full((8, 16)),
        ],
        out_specs=[
            pl.BlockSpec((_NBLK, 16), lambda i: (i, 0)),
            pl.BlockSpec((_NBLK, 16), lambda i: (i, 0)),
            pl.BlockSpec((_NBLK, 1), lambda i: (i, 0)),
            pl.BlockSpec((_NBLK, 1), lambda i: (i, 0)),
        ],
        out_shape=[
            jax.ShapeDtypeStruct((NPAD, 16), F32),
            jax.ShapeDtypeStruct((NPAD, 16), F32),
            jax.ShapeDtypeStruct((NPAD, 1), F32),
            jax.ShapeDtypeStruct((NPAD, 1), F32),
        ],
    )(accp, cntp, xw, w2, b2, w1s, w1d, b1)


def _node_mid_call(i, accp, cnt0, cntc, xw, w2, b2, w1s, w1d, b1):
    full = lambda shape: pl.BlockSpec(shape, lambda i_: tuple(0 for _ in shape))
    return pl.pallas_call(
        functools.partial(_node_mid_kernel, i),
        grid=(_NGRID,),
        in_specs=[
            pl.BlockSpec((NC, _NBLK, 16), lambda i_: (0, i_, 0)),
            pl.BlockSpec((_NBLK, 1), lambda i_: (i_, 0)),
            pl.BlockSpec((_NBLK, 1), lambda i_: (i_, 0)),
            pl.BlockSpec((_NBLK, DN), lambda i_: (i_, 0)),
            full((16, LAT)), full((8, 8)), full((8, 16)), full((8, 16)),
            full((8, 16)),
        ],
        out_specs=[
            pl.BlockSpec((_NBLK, 16), lambda i_: (i_, 0)),
            pl.BlockSpec((_NBLK, 16), lambda i_: (i_, 0)),
        ],
        out_shape=[
            jax.ShapeDtypeStruct((NPAD, 16), F32),
            jax.ShapeDtypeStruct((NPAD, 16), F32),
        ],
    )(accp, cnt0, cntc, xw, w2, b2, w1s, w1d, b1)


def _node_last_call(accp, cnt0, cntc, w2, b2, wf, bf, xlast):
    full = lambda shape: pl.BlockSpec(shape, lambda i: tuple(0 for _ in shape))
    return pl.pallas_call(
        _node_last_kernel,
        grid=(_NGRID,),
        in_specs=[
            pl.BlockSpec((NC, _NBLK, 16), lambda i: (0, i, 0)),
            pl.BlockSpec((_NBLK, 1), lambda i: (i, 0)),
            pl.BlockSpec((_NBLK, 1), lambda i: (i, 0)),
            full((16, LAT)), full((8, 8)), full((8, 1)), full((8, 1)),
            pl.BlockSpec((_NBLK, 1), lambda i: (i, 0)),
        ],
        out_specs=[
            pl.BlockSpec((_NBLK, 1), lambda i: (i, 0)),
            pl.BlockSpec((_NBLK, 1), lambda i: (i, 0)),
        ],
        out_shape=[
            jax.ShapeDtypeStruct((NPAD, 1), F32),
            jax.ShapeDtypeStruct((NPAD, 1), F32),
        ],
    )(accp, cnt0, cntc, w2, b2, wf, bf, xlast)


def _imbal_kernel(netp_ref, p_ref, out_ref):
    net = p_ref[0:1, :] + jnp.sum(netp_ref[...], axis=0, keepdims=True)
    out_ref[...] = jnp.sum(jnp.abs(net)).reshape(1, 1)


def _imbal_call(netp, p_row):
    return pl.pallas_call(
        _imbal_kernel,
        grid=(1,),
        in_specs=[
            pl.BlockSpec((NW, NPAD), lambda i: (0, 0)),
            pl.BlockSpec((8, NPAD), lambda i: (0, 0)),
        ],
        out_specs=pl.BlockSpec((1, 1), lambda i: (0, 0)),
        out_shape=jax.ShapeDtypeStruct((1, 1), F32),
    )(netp, p_row)


def kernel(x, edge_index, edge_attr, W1, b1, W2, b2, Wf, bf):
    src = edge_index[0]
    dst = edge_index[1]
    npad_e = E_PAD - E
    src_p = jnp.concatenate([src, jnp.zeros((npad_e,), jnp.int32)])
    dst_p = jnp.concatenate([dst, jnp.full((npad_e,), N, jnp.int32)])
    src3_p = jnp.concatenate(
        [src_p, jnp.zeros((NSMAX * SUP,), jnp.int32)]).reshape(-1, CHB, 128)
    dst3_p = jnp.concatenate(
        [dst_p, jnp.full((NSMAX * SUP,), N, jnp.int32)]).reshape(-1, CHB, 128)
    ea_p = jnp.concatenate([edge_attr, jnp.zeros((npad_e, DE), F32)], axis=0)
    ea8_p = ea_p.reshape(E_PAD // 8, 8 * DE)
    x_pad = jnp.concatenate([x, jnp.zeros((NPAD - N, DN), F32)], axis=0)
    xlast_pad = x_pad[:, DN - 1:DN]

    w1s = W1[:, 0:LAT, :]
    w1d = W1[:, LAT:2 * LAT, :]
    w1x = W1[:, 2 * LAT:2 * LAT + DN, :]
    w1e = W1[:, 2 * LAT + DN:, :]
    wx_all = jnp.transpose(w1x, (1, 0, 2)).reshape(DN, NCONV * 16)
    we_bd = jax.vmap(
        lambda w: jnp.kron(jnp.eye(8, dtype=F32), w))(w1e)
    w1s_p = jnp.concatenate([w1s, jnp.zeros((NCONV, 2, 16), F32)], axis=1)
    w1d_p = jnp.concatenate([w1d, jnp.zeros((NCONV, 2, 16), F32)], axis=1)
    b1_bc = jnp.broadcast_to(b1[:, None, :], (NCONV, 8, 16))
    b2_bc = jnp.broadcast_to(
        jnp.pad(b2, ((0, 0), (0, 2)))[:, None, :], (NCONV, 8, 8))
    wf_p = jnp.concatenate([Wf, jnp.zeros((2, 1), F32)], axis=0)
    bf_bc = jnp.broadcast_to(bf.reshape(1, 1), (8, 1))

    xw, a0 = _prep_call(x_pad, wx_all, b1_bc[0])
    ews = _ew_call(ea8_p, we_bd)

    edge0 = _make_edge0()
    edge = _make_edge()

    accp, cntp = edge0(a0, ews[0], src3_p, dst3_p)
    a_t, b_t, cnt0, cntc = _node0_call(
        accp, cntp, xw, W2[0], b2_bc[0], w1s_p[1], w1d_p[1], b1_bc[1])

    for i in range(1, NCONV - 1):
        accp = edge(a_t, b_t, ews[i], src3_p, dst3_p)
        a_t, b_t = _node_mid_call(
            i, accp, cnt0, cntc, xw, W2[i], b2_bc[i],
            w1s_p[i + 1], w1d_p[i + 1], b1_bc[i + 1])

    accp = edge(a_t, b_t, ews[NCONV - 1], src3_p, dst3_p)
    p_full, pm_full = _node_last_call(
        accp, cnt0, cntc, W2[NCONV - 1], b2_bc[NCONV - 1], wf_p, bf_bc,
        xlast_pad)

    p_tab = jnp.concatenate([pm_full[:N, 0], jnp.zeros((NPAD - N,), F32)])
    ea0_p = ea_p[:, 0]

    balance = _make_balance()
    flow_pad, netp = balance(p_tab, src_p, dst_p, ea0_p)

    p_row = jnp.broadcast_to(p_tab[None, :], (8, NPAD))
    imb = _imbal_call(netp, p_row)

    P = p_full[:N]
    flow = flow_pad[:E].reshape(E, 1)
    return (P, flow, imb.reshape(1))
